# Initial kernel scaffold; baseline (speedup 1.0000x reference)
import jax
import jax.numpy as jnp
from jax.experimental import pallas as pl


def kernel(edge_index, W1, b1, W2, b2):
    raise NotImplementedError("write your pallas kernel here")


# speedup vs baseline: 4.4204x; 4.4204x over previous
import functools

import jax
import jax.numpy as jnp
from jax import lax
from jax.experimental import pallas as pl
from jax.experimental.pallas import tpu as pltpu
from jax.experimental.pallas import tpu_sc as plsc

N = 10000
E = 320000
H = 128
OUT = 64
NPAD = 10240

NC = 2
NS = 16
NW = NC * NS
RPT = NPAD // NS

EPW = E // NW
ACH = 80
ANCH = EPW // ACH

EPT = E // NS
DCH = 80
DNCH = EPT // DCH

_MESH = plsc.VectorSubcoreMesh(core_axis_name="c", subcore_axis_name="s")


def _deg_body(src_hbm, dst_hbm, zeros_hbm, dsrc_hbm, ddst_hbm,
              idx_v, ones_v, acc, sem):
  cid = lax.axis_index("c")
  sid = lax.axis_index("s")
  for k in range(DCH // 16):
    ones_v[pl.ds(k * 16, 16)] = jnp.ones((16,), jnp.float32)
  r0 = pl.multiple_of(sid * RPT, 8)
  pltpu.sync_copy(zeros_hbm.at[pl.ds(r0, RPT)], acc.at[pl.ds(r0, RPT)])
  plsc.subcore_barrier()

  base = sid * EPT

  def body(i, _):
    off = pl.multiple_of(base + i * DCH, 8)

    @pl.when(cid == 0)
    def _():
      pltpu.sync_copy(src_hbm.at[pl.ds(off, DCH)], idx_v)

    @pl.when(cid == 1)
    def _():
      pltpu.sync_copy(dst_hbm.at[pl.ds(off, DCH)], idx_v)

    pltpu.sync_copy(ones_v, acc.at[idx_v], add=True)
    return _

  lax.fori_loop(0, DNCH, body, None)
  plsc.subcore_barrier()

  @pl.when(cid == 0)
  def _():
    pltpu.sync_copy(acc.at[pl.ds(r0, RPT)], dsrc_hbm.at[pl.ds(r0, RPT)])

  @pl.when(cid == 1)
  def _():
    pltpu.sync_copy(acc.at[pl.ds(r0, RPT)], ddst_hbm.at[pl.ds(r0, RPT)])


_deg_kernel = functools.partial(
    pl.kernel,
    out_type=[jax.ShapeDtypeStruct((NPAD,), jnp.float32),
              jax.ShapeDtypeStruct((NPAD,), jnp.float32)],
    mesh=_MESH,
    scratch_types=[
        pltpu.VMEM((DCH,), jnp.int32),
        pltpu.VMEM((DCH,), jnp.float32),
        pltpu.VMEM_SHARED((NPAD,), jnp.float32),
        pltpu.SemaphoreType.DMA,
    ],
)(_deg_body)


def _make_agg(F):
  def body(hw_hbm, src_hbm, dst_hbm, zeros_hbm, out_hbm,
           sidx, didx, rows, acc, sem):
    cid = lax.axis_index("c")
    sid = lax.axis_index("s")
    wid = cid * NS + sid
    r0 = pl.multiple_of(sid * RPT, 8)
    pltpu.sync_copy(zeros_hbm.at[pl.ds(r0, RPT)], acc.at[pl.ds(r0, RPT)])
    plsc.subcore_barrier()

    base = wid * EPW

    def step(i, _):
      off = pl.multiple_of(base + i * ACH, 8)
      pltpu.sync_copy(src_hbm.at[pl.ds(off, ACH)], sidx)
      pltpu.sync_copy(dst_hbm.at[pl.ds(off, ACH)], didx)
      pltpu.async_copy(hw_hbm.at[sidx], rows, sem).wait()
      pltpu.sync_copy(rows, acc.at[didx], add=True)
      return _

    lax.fori_loop(0, ANCH, step, None)
    plsc.subcore_barrier()
    pltpu.sync_copy(acc.at[pl.ds(r0, RPT)],
                    out_hbm.at[cid].at[pl.ds(r0, RPT)])

  return pl.kernel(
      body,
      out_type=jax.ShapeDtypeStruct((NC, NPAD, F), jnp.float32),
      mesh=_MESH,
      scratch_types=[
          pltpu.VMEM((ACH,), jnp.int32),
          pltpu.VMEM((ACH,), jnp.int32),
          pltpu.VMEM((ACH, F), jnp.float32),
          pltpu.VMEM_SHARED((NPAD, F), jnp.float32),
          pltpu.SemaphoreType.DMA,
      ],
  )


_agg_h = _make_agg(H)


def _norms_body(dsrc_ref, ddst_ref, ns_ref, nd_ref):
  ns_ref[...] = lax.rsqrt(jnp.maximum(dsrc_ref[...], 1.0))
  nd_ref[...] = lax.rsqrt(jnp.maximum(ddst_ref[...], 1.0))


_norms_tc = pl.pallas_call(
    _norms_body,
    out_shape=[jax.ShapeDtypeStruct((NPAD,), jnp.float32),
               jax.ShapeDtypeStruct((NPAD,), jnp.float32)],
)


def _first_body(ns_ref, w_ref, out_ref):
  out_ref[...] = jnp.zeros((NPAD, H), jnp.float32)
  out_ref[0:H, :] = w_ref[...] * ns_ref[0:H]


_first_tc = pl.pallas_call(
    _first_body,
    out_shape=jax.ShapeDtypeStruct((NPAD, H), jnp.float32),
)


def _make_mid(K):
  def body(agg_ref, nd_ref, b_ref, ns_ref, w_ref, out_ref):
    a = agg_ref[0] + agg_ref[1]
    y = jnp.maximum(a * nd_ref[...] + b_ref[...], 0.0)
    out_ref[...] = jnp.dot(y * ns_ref[...], w_ref[...],
                           preferred_element_type=jnp.float32)

  return pl.pallas_call(
      body, out_shape=jax.ShapeDtypeStruct((NPAD, K), jnp.float32))


_mid_tc_h = _make_mid(H)


def _final_body(agg_ref, nd_ref, b_ref, out_ref):
  a = agg_ref[0, :, 0:OUT] + agg_ref[1, :, 0:OUT]
  out_ref[...] = a * nd_ref[...] + b_ref[...]


_final_tc = pl.pallas_call(
    _final_body,
    out_shape=jax.ShapeDtypeStruct((NPAD, OUT), jnp.float32),
)


def kernel(edge_index, W1, b1, W2, b2):
  src = edge_index[0].astype(jnp.int32)
  dst = edge_index[1].astype(jnp.int32)
  zeros_nh = jnp.zeros((NPAD, H), jnp.float32)
  zeros_n = jnp.zeros((NPAD,), jnp.float32)
  W2p = jnp.pad(W2, ((0, 0), (0, H - OUT)))

  dsrc, ddst = _deg_kernel(src, dst, zeros_n)
  ns, nd = _norms_tc(dsrc, ddst)
  ns2 = ns.reshape(NPAD, 1)
  nd2 = nd.reshape(NPAD, 1)
  b1r = b1.reshape(1, H)
  b2r = b2.reshape(1, OUT)

  hw = _first_tc(ns2, W1)
  agg = _agg_h(hw, src, dst, zeros_nh)
  hw = _mid_tc_h(agg, nd2, b1r, ns2, W1)
  agg = _agg_h(hw, src, dst, zeros_nh)
  hw = _mid_tc_h(agg, nd2, b1r, ns2, W1)
  agg = _agg_h(hw, src, dst, zeros_nh)
  hw2 = _mid_tc_h(agg, nd2, b1r, ns2, W2p)
  agg2 = _agg_h(hw2, src, dst, zeros_nh)
  out = _final_tc(agg2, nd2, b2r)
  return out[:N]


# speedup vs baseline: 6.9863x; 1.5805x over previous
import functools

import jax
import jax.numpy as jnp
from jax import lax
from jax.experimental import pallas as pl
from jax.experimental.pallas import tpu as pltpu
from jax.experimental.pallas import tpu_sc as plsc

N = 10000
E = 320000
H = 128
OUT = 64
NPAD = 10240

NC = 2
NS = 16
NW = NC * NS
RPT = NPAD // NS

EPW = E // NW
ACH = 40
ANCH = EPW // ACH
ASB = 25
ANSB = ANCH // ASB

EPT = E // NS
DCH = 80
CW = NPAD * H + 2048
CPT = NPAD * H // NS

_MESH = plsc.VectorSubcoreMesh(core_axis_name="c", subcore_axis_name="s")


def _prep_body(sd_hbm, zdeg_hbm, zc_hbm, degs_hbm, csum_hbm,
               degidx_v, csrc_v, cdst_v, cidx_v, ones_v, dacc, cacc, sem):
  cid = lax.axis_index("c")
  sid = lax.axis_index("s")
  for k in range(DCH // 16):
    ones_v[pl.ds(k * 16, 16)] = jnp.ones((16,), jnp.float32)

  r0 = pl.multiple_of(sid * RPT, 8)
  z0 = pl.multiple_of(sid * (CW // NS), 8)
  pltpu.sync_copy(zdeg_hbm.at[pl.ds(r0, RPT)], dacc.at[pl.ds(r0, RPT)])
  pltpu.sync_copy(zc_hbm.at[pl.ds(z0, CW // NS)], cacc.at[pl.ds(z0, CW // NS)])
  plsc.subcore_barrier()

  doff = pl.multiple_of(cid * E + sid * EPT, 8)
  wid = cid * NS + sid
  coff = pl.multiple_of(wid * EPW, 8)

  def dstep(i, _):
    off = pl.multiple_of(doff + i * DCH, 8)
    pltpu.sync_copy(sd_hbm.at[pl.ds(off, DCH)], degidx_v)
    pltpu.sync_copy(ones_v, dacc.at[degidx_v], add=True)
    return _

  lax.fori_loop(0, EPT // DCH, dstep, None)

  def cstep(i, _):
    soff = pl.multiple_of(coff + i * DCH, 8)
    pltpu.sync_copy(sd_hbm.at[pl.ds(soff, DCH)], csrc_v)
    pltpu.sync_copy(sd_hbm.at[pl.ds(soff + E, DCH)], cdst_v)
    for k in range(DCH // 16):
      sl = pl.ds(k * 16, 16)
      s = csrc_v[sl]
      d = cdst_v[sl]
      cidx_v[sl] = jnp.where(s < H, (d << 7) + s, (NPAD * H) + (s & 127))
    pltpu.sync_copy(ones_v, cacc.at[cidx_v], add=True)
    return _

  lax.fori_loop(0, EPW // DCH, cstep, None)
  plsc.subcore_barrier()

  pltpu.sync_copy(dacc.at[pl.ds(r0, RPT)],
                  degs_hbm.at[pl.ds(pl.multiple_of(cid * NPAD + r0, 8), RPT)])
  cd0 = pl.multiple_of(sid * CPT, 8)
  pltpu.sync_copy(
      cacc.at[pl.ds(cd0, CPT)],
      csum_hbm.at[pl.ds(pl.multiple_of(cid * (NPAD * H) + cd0, 8), CPT)])


_prep_kernel = functools.partial(
    pl.kernel,
    out_type=[jax.ShapeDtypeStruct((2 * NPAD,), jnp.float32),
              jax.ShapeDtypeStruct((2 * NPAD * H,), jnp.float32)],
    mesh=_MESH,
    scratch_types=[
        pltpu.VMEM((DCH,), jnp.int32),
        pltpu.VMEM((DCH,), jnp.int32),
        pltpu.VMEM((DCH,), jnp.int32),
        pltpu.VMEM((DCH,), jnp.int32),
        pltpu.VMEM((DCH,), jnp.float32),
        pltpu.VMEM_SHARED((NPAD,), jnp.float32),
        pltpu.VMEM_SHARED((CW,), jnp.float32),
        pltpu.SemaphoreType.DMA,
    ],
)(_prep_body)


def _make_agg(F):
  def body(hw_hbm, src_hbm, dst_hbm, zeros_hbm, out_hbm,
           sstage, dstage, didx0, didx1, rows0, rows1, acc,
           sg0, sg1, ss0, ss1):
    cid = lax.axis_index("c")
    sid = lax.axis_index("s")
    wid = cid * NS + sid
    r0 = pl.multiple_of(sid * RPT, 8)
    pltpu.sync_copy(zeros_hbm.at[pl.ds(r0, RPT)], acc.at[pl.ds(r0, RPT)])
    plsc.subcore_barrier()
    base = wid * EPW

    def vcopy(dref, j):
      for o in (0, 16, ACH - 16):
        dref[pl.ds(o, 16)] = dstage[pl.ds(j * ACH + o, 16)]

    def g_desc(j, rbuf, sem):
      return pltpu.make_async_copy(
          hw_hbm.at[sstage.at[pl.ds(j * ACH, ACH)]], rbuf, sem)

    def s_desc(rbuf, dref, sem):
      return pltpu.make_async_copy(rbuf, acc.at[dref], sem)

    def sblock(sb, _):
      off = pl.multiple_of(base + sb * (ASB * ACH), 8)
      pltpu.sync_copy(src_hbm.at[pl.ds(off, ASB * ACH)], sstage)
      pltpu.sync_copy(dst_hbm.at[pl.ds(off, ASB * ACH)], dstage)
      vcopy(didx0, 0)
      g_desc(0, rows0, sg0).start()

      def pair(t, _):
        j0 = 2 * t
        j1 = j0 + 1
        g_desc(j1, rows1, sg1).start()
        g_desc(j0, rows0, sg0).wait()

        @pl.when(t > 0)
        def _():
          s_desc(rows1, didx1, ss1).wait()

        vcopy(didx1, j1)
        pltpu.async_copy(rows0, acc.at[didx0], ss0, add=True)
        g_desc(j1, rows1, sg1).wait()
        s_desc(rows0, didx0, ss0).wait()
        vcopy(didx0, j0 + 2)
        g_desc(j0 + 2, rows0, sg0).start()
        pltpu.async_copy(rows1, acc.at[didx1], ss1, add=True)
        return _

      lax.fori_loop(0, ASB // 2, pair, None)
      g_desc(ASB - 1, rows0, sg0).wait()
      s_desc(rows1, didx1, ss1).wait()
      pltpu.async_copy(rows0, acc.at[didx0], ss0, add=True)
      s_desc(rows0, didx0, ss0).wait()
      return _

    lax.fori_loop(0, ANSB, sblock, None)
    plsc.subcore_barrier()
    pltpu.sync_copy(acc.at[pl.ds(r0, RPT)],
                    out_hbm.at[cid].at[pl.ds(r0, RPT)])

  return pl.kernel(
      body,
      out_type=jax.ShapeDtypeStruct((NC, NPAD, F), jnp.float32),
      mesh=_MESH,
      scratch_types=[
          pltpu.VMEM((ASB * ACH,), jnp.int32),
          pltpu.VMEM((ASB * ACH,), jnp.int32),
          pltpu.VMEM((ACH,), jnp.int32),
          pltpu.VMEM((ACH,), jnp.int32),
          pltpu.VMEM((ACH, F), jnp.float32),
          pltpu.VMEM((ACH, F), jnp.float32),
          pltpu.VMEM_SHARED((NPAD, F), jnp.float32),
          pltpu.SemaphoreType.DMA,
          pltpu.SemaphoreType.DMA,
          pltpu.SemaphoreType.DMA,
          pltpu.SemaphoreType.DMA,
      ],
  )


_agg_h = _make_agg(H)


def _norms_body(dsrc_ref, ddst_ref, ns_ref, nd_ref):
  ns_ref[...] = lax.rsqrt(jnp.maximum(dsrc_ref[...], 1.0))
  nd_ref[...] = lax.rsqrt(jnp.maximum(ddst_ref[...], 1.0))


_norms_tc = pl.pallas_call(
    _norms_body,
    out_shape=[jax.ShapeDtypeStruct((NPAD,), jnp.float32),
               jax.ShapeDtypeStruct((NPAD,), jnp.float32)],
)


def _l1mid_body(c0_ref, c1_ref, nd_ref, b_ref, ns_ref, w_ref, out_ref):
  c = c0_ref[...] + c1_ref[...]
  w1s = w_ref[...] * ns_ref[0:H]
  agg = jnp.dot(c, w1s, preferred_element_type=jnp.float32)
  y = jnp.maximum(agg * nd_ref[...] + b_ref[...], 0.0)
  out_ref[...] = jnp.dot(y * ns_ref[...], w_ref[...],
                         preferred_element_type=jnp.float32)


_l1mid_tc = pl.pallas_call(
    _l1mid_body,
    out_shape=jax.ShapeDtypeStruct((NPAD, H), jnp.float32),
)


def _make_mid(K):
  def body(agg_ref, nd_ref, b_ref, ns_ref, w_ref, out_ref):
    a = agg_ref[0] + agg_ref[1]
    y = jnp.maximum(a * nd_ref[...] + b_ref[...], 0.0)
    out_ref[...] = jnp.dot(y * ns_ref[...], w_ref[...],
                           preferred_element_type=jnp.float32)

  return pl.pallas_call(
      body, out_shape=jax.ShapeDtypeStruct((NPAD, K), jnp.float32))


_mid_tc_h = _make_mid(H)


def _final_body(agg_ref, nd_ref, b_ref, out_ref):
  a = agg_ref[0, :, 0:OUT] + agg_ref[1, :, 0:OUT]
  out_ref[...] = a * nd_ref[...] + b_ref[...]


_final_tc = pl.pallas_call(
    _final_body,
    out_shape=jax.ShapeDtypeStruct((NPAD, OUT), jnp.float32),
)


def kernel(edge_index, W1, b1, W2, b2):
  src = edge_index[0].astype(jnp.int32)
  dst = edge_index[1].astype(jnp.int32)
  zeros_nh = jnp.zeros((NPAD, H), jnp.float32)
  zeros_n = jnp.zeros((NPAD,), jnp.float32)
  zeros_c = jnp.zeros((CW,), jnp.float32)
  W2p = jnp.pad(W2, ((0, 0), (0, H - OUT)))

  sd = jnp.concatenate([src, dst])
  degs, csum = _prep_kernel(sd, zeros_n, zeros_c)
  dsrc = degs[:NPAD]
  ddst = degs[NPAD:]
  c0 = csum[:NPAD * H].reshape(NPAD, H)
  c1 = csum[NPAD * H:].reshape(NPAD, H)
  ns, nd = _norms_tc(dsrc, ddst)
  ns2 = ns.reshape(NPAD, 1)
  nd2 = nd.reshape(NPAD, 1)
  b1r = b1.reshape(1, H)
  b2r = b2.reshape(1, OUT)

  hw = _l1mid_tc(c0, c1, nd2, b1r, ns2, W1)
  agg = _agg_h(hw, src, dst, zeros_nh)
  hw = _mid_tc_h(agg, nd2, b1r, ns2, W1)
  agg = _agg_h(hw, src, dst, zeros_nh)
  hw2 = _mid_tc_h(agg, nd2, b1r, ns2, W2p)
  agg2 = _agg_h(hw2, src, dst, zeros_nh)
  out = _final_tc(agg2, nd2, b2r)
  return out[:N]


# speedup vs baseline: 7.4412x; 1.0651x over previous
import functools

import jax
import jax.numpy as jnp
from jax import lax
from jax.experimental import pallas as pl
from jax.experimental.pallas import tpu as pltpu
from jax.experimental.pallas import tpu_sc as plsc

N = 10000
E = 320000
H = 128
OUT = 64
NPAD = 10240

NC = 2
NS = 16
NW = NC * NS
RPT = NPAD // NS

EPW = E // NW
ACH = 80
ANCH = EPW // ACH
ASB = 5
ANSB = ANCH // ASB

EPT = E // NS
DCH = 80
CW = NPAD * H + 2048
CPT = NPAD * H // NS

_MESH = plsc.VectorSubcoreMesh(core_axis_name="c", subcore_axis_name="s")


def _prep_body(sd_hbm, zdeg_hbm, zc_hbm, degs_hbm, csum_hbm,
               di0, di1, csrc_v, cdst_v, ci0, ci1, ones_v, dacc, cacc,
               si0, si1, ss0, ss1):
  cid = lax.axis_index("c")
  sid = lax.axis_index("s")
  for k in range(DCH // 16):
    ones_v[pl.ds(k * 16, 16)] = jnp.ones((16,), jnp.float32)

  r0 = pl.multiple_of(sid * RPT, 8)
  z0 = pl.multiple_of(sid * (CW // NS), 8)
  pltpu.sync_copy(zdeg_hbm.at[pl.ds(r0, RPT)], dacc.at[pl.ds(r0, RPT)])
  pltpu.sync_copy(zc_hbm.at[pl.ds(z0, CW // NS)], cacc.at[pl.ds(z0, CW // NS)])
  plsc.subcore_barrier()

  doff = pl.multiple_of(cid * E + sid * EPT, 8)
  wid = cid * NS + sid
  coff = pl.multiple_of(wid * EPW, 8)

  def dload(j, buf, sem):
    off = pl.multiple_of(doff + j * DCH, 8)
    return pltpu.make_async_copy(sd_hbm.at[pl.ds(off, DCH)], buf, sem)

  def dscat(buf, sem):
    return pltpu.make_async_copy(ones_v, dacc.at[buf], sem)

  DP = EPT // DCH // 2

  dload(0, di0, si0).start()

  def dpair(t, _):
    j0 = 2 * t
    dload(j0, di0, si0).wait()

    @pl.when(t > 0)
    def _():
      dscat(di1, ss1).wait()
    dload(j0 + 1, di1, si1).start()
    pltpu.async_copy(ones_v, dacc.at[di0], ss0, add=True)
    dload(j0 + 1, di1, si1).wait()
    dscat(di0, ss0).wait()

    @pl.when(t + 1 < DP)
    def _():
      dload(j0 + 2, di0, si0).start()
    pltpu.async_copy(ones_v, dacc.at[di1], ss1, add=True)
    return _

  lax.fori_loop(0, DP, dpair, None)
  dscat(di1, ss1).wait()

  def cload(j, _):
    soff = pl.multiple_of(coff + j * DCH, 8)
    pltpu.sync_copy(sd_hbm.at[pl.ds(soff, DCH)], csrc_v)
    pltpu.sync_copy(sd_hbm.at[pl.ds(soff + E, DCH)], cdst_v)

  def cfill(iref):
    for k in range(DCH // 16):
      sl = pl.ds(k * 16, 16)
      s = csrc_v[sl]
      d = cdst_v[sl]
      iref[sl] = jnp.where(s < H, (d << 7) + s, (NPAD * H) + (s & 127))

  def cscat(buf, sem):
    return pltpu.make_async_copy(ones_v, cacc.at[buf], sem)

  CP = EPW // DCH // 2

  cload(0, None)
  cfill(ci0)

  def cpair(t, _):
    j0 = 2 * t
    cload(j0 + 1, None)

    @pl.when(t > 0)
    def _():
      cscat(ci1, ss1).wait()
    pltpu.async_copy(ones_v, cacc.at[ci0], ss0, add=True)
    cfill(ci1)
    cload(j0 + 2, None)
    cscat(ci0, ss0).wait()
    pltpu.async_copy(ones_v, cacc.at[ci1], ss1, add=True)
    cfill(ci0)
    return _

  lax.fori_loop(0, CP, cpair, None)
  cscat(ci1, ss1).wait()
  pltpu.async_copy(ones_v, cacc.at[ci0], ss0, add=True)
  cscat(ci0, ss0).wait()
  plsc.subcore_barrier()

  pltpu.sync_copy(dacc.at[pl.ds(r0, RPT)],
                  degs_hbm.at[pl.ds(pl.multiple_of(cid * NPAD + r0, 8), RPT)])
  cd0 = pl.multiple_of(sid * CPT, 8)
  pltpu.sync_copy(
      cacc.at[pl.ds(cd0, CPT)],
      csum_hbm.at[pl.ds(pl.multiple_of(cid * (NPAD * H) + cd0, 8), CPT)])


_prep_kernel = functools.partial(
    pl.kernel,
    out_type=[jax.ShapeDtypeStruct((2 * NPAD,), jnp.float32),
              jax.ShapeDtypeStruct((2 * NPAD * H,), jnp.float32)],
    mesh=_MESH,
    scratch_types=[
        pltpu.VMEM((DCH,), jnp.int32),
        pltpu.VMEM((DCH,), jnp.int32),
        pltpu.VMEM((DCH,), jnp.int32),
        pltpu.VMEM((DCH,), jnp.int32),
        pltpu.VMEM((DCH,), jnp.int32),
        pltpu.VMEM((DCH,), jnp.int32),
        pltpu.VMEM((DCH,), jnp.float32),
        pltpu.VMEM_SHARED((NPAD,), jnp.float32),
        pltpu.VMEM_SHARED((CW,), jnp.float32),
        pltpu.SemaphoreType.DMA,
        pltpu.SemaphoreType.DMA,
        pltpu.SemaphoreType.DMA,
        pltpu.SemaphoreType.DMA,
    ],
)(_prep_body)


def _make_agg(F):
  def body(hw_hbm, src_hbm, dst_hbm, zeros_hbm, out_hbm,
           sstage, dstage, didx0, didx1, rows0, rows1, acc,
           sg0, sg1, ss0, ss1):
    cid = lax.axis_index("c")
    sid = lax.axis_index("s")
    wid = cid * NS + sid
    r0 = pl.multiple_of(sid * RPT, 8)
    pltpu.sync_copy(zeros_hbm.at[pl.ds(r0, RPT)], acc.at[pl.ds(r0, RPT)])
    plsc.subcore_barrier()
    base = wid * EPW

    def vcopy(dref, j):
      for o in sorted(set(list(range(0, ACH - 15, 16)) + [ACH - 16])):
        dref[pl.ds(o, 16)] = dstage[pl.ds(j * ACH + o, 16)]

    def g_desc(j, rbuf, sem):
      return pltpu.make_async_copy(
          hw_hbm.at[sstage.at[pl.ds(j * ACH, ACH)]], rbuf, sem)

    def s_desc(rbuf, dref, sem):
      return pltpu.make_async_copy(rbuf, acc.at[dref], sem)

    def sblock(sb, _):
      off = pl.multiple_of(base + sb * (ASB * ACH), 8)
      pltpu.sync_copy(src_hbm.at[pl.ds(off, ASB * ACH)], sstage)
      pltpu.sync_copy(dst_hbm.at[pl.ds(off, ASB * ACH)], dstage)
      vcopy(didx0, 0)
      g_desc(0, rows0, sg0).start()

      def pair(t, _):
        j0 = 2 * t
        j1 = j0 + 1
        g_desc(j1, rows1, sg1).start()
        g_desc(j0, rows0, sg0).wait()

        @pl.when(t > 0)
        def _():
          s_desc(rows1, didx1, ss1).wait()

        vcopy(didx1, j1)
        pltpu.async_copy(rows0, acc.at[didx0], ss0, add=True)
        g_desc(j1, rows1, sg1).wait()
        s_desc(rows0, didx0, ss0).wait()
        vcopy(didx0, j0 + 2)
        g_desc(j0 + 2, rows0, sg0).start()
        pltpu.async_copy(rows1, acc.at[didx1], ss1, add=True)
        return _

      lax.fori_loop(0, ASB // 2, pair, None)
      g_desc(ASB - 1, rows0, sg0).wait()
      s_desc(rows1, didx1, ss1).wait()
      pltpu.async_copy(rows0, acc.at[didx0], ss0, add=True)
      s_desc(rows0, didx0, ss0).wait()
      return _

    lax.fori_loop(0, ANSB, sblock, None)
    plsc.subcore_barrier()
    pltpu.sync_copy(acc.at[pl.ds(r0, RPT)],
                    out_hbm.at[cid].at[pl.ds(r0, RPT)])

  return pl.kernel(
      body,
      out_type=jax.ShapeDtypeStruct((NC, NPAD, F), jnp.float32),
      mesh=_MESH,
      scratch_types=[
          pltpu.VMEM((ASB * ACH,), jnp.int32),
          pltpu.VMEM((ASB * ACH,), jnp.int32),
          pltpu.VMEM((ACH,), jnp.int32),
          pltpu.VMEM((ACH,), jnp.int32),
          pltpu.VMEM((ACH, F), jnp.float32),
          pltpu.VMEM((ACH, F), jnp.float32),
          pltpu.VMEM_SHARED((NPAD, F), jnp.float32),
          pltpu.SemaphoreType.DMA,
          pltpu.SemaphoreType.DMA,
          pltpu.SemaphoreType.DMA,
          pltpu.SemaphoreType.DMA,
      ],
  )


_agg_h = _make_agg(H)


def _norms_body(dsrc_ref, ddst_ref, ns_ref, nd_ref):
  ns_ref[...] = lax.rsqrt(jnp.maximum(dsrc_ref[...], 1.0))
  nd_ref[...] = lax.rsqrt(jnp.maximum(ddst_ref[...], 1.0))


_norms_tc = pl.pallas_call(
    _norms_body,
    out_shape=[jax.ShapeDtypeStruct((NPAD,), jnp.float32),
               jax.ShapeDtypeStruct((NPAD,), jnp.float32)],
)


def _l1mid_body(c0_ref, c1_ref, nd_ref, b_ref, ns_ref, w_ref, out_ref):
  c = c0_ref[...] + c1_ref[...]
  w1s = w_ref[...] * ns_ref[0:H]
  agg = jnp.dot(c, w1s, preferred_element_type=jnp.float32)
  y = jnp.maximum(agg * nd_ref[...] + b_ref[...], 0.0)
  out_ref[...] = jnp.dot(y * ns_ref[...], w_ref[...],
                         preferred_element_type=jnp.float32)


_l1mid_tc = pl.pallas_call(
    _l1mid_body,
    out_shape=jax.ShapeDtypeStruct((NPAD, H), jnp.float32),
)


def _make_mid(K):
  def body(agg_ref, nd_ref, b_ref, ns_ref, w_ref, out_ref):
    a = agg_ref[0] + agg_ref[1]
    y = jnp.maximum(a * nd_ref[...] + b_ref[...], 0.0)
    out_ref[...] = jnp.dot(y * ns_ref[...], w_ref[...],
                           preferred_element_type=jnp.float32)

  return pl.pallas_call(
      body, out_shape=jax.ShapeDtypeStruct((NPAD, K), jnp.float32))


_mid_tc_h = _make_mid(H)


def _final_body(agg_ref, nd_ref, b_ref, out_ref):
  a = agg_ref[0, :, 0:OUT] + agg_ref[1, :, 0:OUT]
  out_ref[...] = a * nd_ref[...] + b_ref[...]


_final_tc = pl.pallas_call(
    _final_body,
    out_shape=jax.ShapeDtypeStruct((NPAD, OUT), jnp.float32),
)


def kernel(edge_index, W1, b1, W2, b2):
  src = edge_index[0].astype(jnp.int32)
  dst = edge_index[1].astype(jnp.int32)
  zeros_nh = jnp.zeros((NPAD, H), jnp.float32)
  zeros_n = jnp.zeros((NPAD,), jnp.float32)
  zeros_c = jnp.zeros((CW,), jnp.float32)
  W2p = jnp.pad(W2, ((0, 0), (0, H - OUT)))

  sd = jnp.concatenate([src, dst])
  degs, csum = _prep_kernel(sd, zeros_n, zeros_c)
  dsrc = degs[:NPAD]
  ddst = degs[NPAD:]
  c0 = csum[:NPAD * H].reshape(NPAD, H)
  c1 = csum[NPAD * H:].reshape(NPAD, H)
  ns, nd = _norms_tc(dsrc, ddst)
  ns2 = ns.reshape(NPAD, 1)
  nd2 = nd.reshape(NPAD, 1)
  b1r = b1.reshape(1, H)
  b2r = b2.reshape(1, OUT)

  hw = _l1mid_tc(c0, c1, nd2, b1r, ns2, W1)
  agg = _agg_h(hw, src, dst, zeros_nh)
  hw = _mid_tc_h(agg, nd2, b1r, ns2, W1)
  agg = _agg_h(hw, src, dst, zeros_nh)
  hw2 = _mid_tc_h(agg, nd2, b1r, ns2, W2p)
  agg2 = _agg_h(hw2, src, dst, zeros_nh)
  out = _final_tc(agg2, nd2, b2r)
  return out[:N]


# speedup vs baseline: 8.1394x; 1.0938x over previous
import functools

import jax
import jax.numpy as jnp
from jax import lax
from jax.experimental import pallas as pl
from jax.experimental.pallas import tpu as pltpu
from jax.experimental.pallas import tpu_sc as plsc

N = 10000
E = 320000
H = 128
OUT = 64
NPAD = 10240

NC = 2
NS = 16
NW = NC * NS
RPT = NPAD // NS

EPW = E // NW
ACH = 80
ANCH = EPW // ACH
ASB = 5
ANSB = ANCH // ASB

EPT = E // NS
DCH = 80
DBLK = 2000
CW = NPAD * H + 2048
CPT = NPAD * H // NS

_MESH = plsc.VectorSubcoreMesh(core_axis_name="c", subcore_axis_name="s")


def _prep_body(sd_hbm, zc_hbm, degs_hbm, csum_hbm,
               dstage_v, hist_v, part_v, csrc_v, cdst_v, ci0, ci1, ones_v,
               hstage, cacc, si0, si1, ss0, ss1):
  cid = lax.axis_index("c")
  sid = lax.axis_index("s")
  for k in range(DCH // 16):
    ones_v[pl.ds(k * 16, 16)] = jnp.ones((16,), jnp.float32)

  r0 = pl.multiple_of(sid * RPT, 8)
  z0 = pl.multiple_of(sid * (CW // NS), 8)
  pltpu.sync_copy(zc_hbm.at[pl.ds(z0, CW // NS)], cacc.at[pl.ds(z0, CW // NS)])

  doff = pl.multiple_of(cid * E + sid * EPT, 8)
  wid = cid * NS + sid
  coff = pl.multiple_of(wid * EPW, 8)

  def hzero(i, _):
    hist_v[pl.ds(i * 16, 16)] = jnp.zeros((16,), jnp.int32)
    return _

  lax.fori_loop(0, NPAD // 16, hzero, None)

  def dblk(blk, _):
    off = pl.multiple_of(doff + blk * DBLK, 8)
    pltpu.sync_copy(sd_hbm.at[pl.ds(off, DBLK)], dstage_v)

    def dgrp(g, _):
      idx = dstage_v[pl.ds(g * 16, 16)]
      cnt, last = plsc.scan_count(idx)
      plsc.addupdate_scatter(hist_v, [idx], cnt, mask=last)
      return _

    lax.fori_loop(0, DBLK // 16, dgrp, None)
    return _

  lax.fori_loop(0, EPT // DBLK, dblk, None)

  pltpu.sync_copy(hist_v, hstage.at[sid])
  plsc.subcore_barrier()
  pltpu.sync_copy(hstage.at[0].at[pl.ds(r0, RPT)], hist_v.at[pl.ds(0, RPT)])
  for b in range(1, NS):
    pltpu.sync_copy(hstage.at[b].at[pl.ds(r0, RPT)], part_v)
    for g in range(RPT // 16):
      sl = pl.ds(g * 16, 16)
      hist_v[sl] = hist_v[sl] + part_v[sl]
  pltpu.sync_copy(hist_v.at[pl.ds(0, RPT)],
                  degs_hbm.at[pl.ds(pl.multiple_of(cid * NPAD + r0, 8), RPT)])

  def cload(j, _):
    soff = pl.multiple_of(coff + j * DCH, 8)
    pltpu.sync_copy(sd_hbm.at[pl.ds(soff, DCH)], csrc_v)
    pltpu.sync_copy(sd_hbm.at[pl.ds(soff + E, DCH)], cdst_v)

  def cfill(iref):
    for k in range(DCH // 16):
      sl = pl.ds(k * 16, 16)
      s = csrc_v[sl]
      d = cdst_v[sl]
      iref[sl] = jnp.where(s < H, (d << 7) + s, (NPAD * H) + (s & 127))

  def cscat(buf, sem):
    return pltpu.make_async_copy(ones_v, cacc.at[buf], sem)

  CP = EPW // DCH // 2

  cload(0, None)
  cfill(ci0)

  def cpair(t, _):
    j0 = 2 * t
    cload(j0 + 1, None)

    @pl.when(t > 0)
    def _():
      cscat(ci1, ss1).wait()
    pltpu.async_copy(ones_v, cacc.at[ci0], ss0, add=True)
    cfill(ci1)
    cload(j0 + 2, None)
    cscat(ci0, ss0).wait()
    pltpu.async_copy(ones_v, cacc.at[ci1], ss1, add=True)
    cfill(ci0)
    return _

  lax.fori_loop(0, CP, cpair, None)
  cscat(ci1, ss1).wait()
  pltpu.async_copy(ones_v, cacc.at[ci0], ss0, add=True)
  cscat(ci0, ss0).wait()
  plsc.subcore_barrier()

  cd0 = pl.multiple_of(sid * CPT, 8)
  pltpu.sync_copy(
      cacc.at[pl.ds(cd0, CPT)],
      csum_hbm.at[pl.ds(pl.multiple_of(cid * (NPAD * H) + cd0, 8), CPT)])


_prep_kernel = functools.partial(
    pl.kernel,
    out_type=[jax.ShapeDtypeStruct((2 * NPAD,), jnp.int32),
              jax.ShapeDtypeStruct((2 * NPAD * H,), jnp.float32)],
    mesh=_MESH,
    compiler_params=pltpu.CompilerParams(needs_layout_passes=False),
    scratch_types=[
        pltpu.VMEM((DBLK,), jnp.int32),
        pltpu.VMEM((NPAD,), jnp.int32),
        pltpu.VMEM((RPT,), jnp.int32),
        pltpu.VMEM((DCH,), jnp.int32),
        pltpu.VMEM((DCH,), jnp.int32),
        pltpu.VMEM((DCH,), jnp.int32),
        pltpu.VMEM((DCH,), jnp.int32),
        pltpu.VMEM((DCH,), jnp.float32),
        pltpu.VMEM_SHARED((NS, NPAD), jnp.int32),
        pltpu.VMEM_SHARED((CW,), jnp.float32),
        pltpu.SemaphoreType.DMA,
        pltpu.SemaphoreType.DMA,
        pltpu.SemaphoreType.DMA,
        pltpu.SemaphoreType.DMA,
    ],
)(_prep_body)


def _make_agg(F):
  def body(hw_hbm, src_hbm, dst_hbm, zeros_hbm, out_hbm,
           sstage, dstage, didx0, didx1, rows0, rows1, acc,
           sg0, sg1, ss0, ss1):
    cid = lax.axis_index("c")
    sid = lax.axis_index("s")
    wid = cid * NS + sid
    r0 = pl.multiple_of(sid * RPT, 8)
    pltpu.sync_copy(zeros_hbm.at[pl.ds(r0, RPT)], acc.at[pl.ds(r0, RPT)])
    plsc.subcore_barrier()
    base = wid * EPW

    def vcopy(dref, j):
      for o in sorted(set(list(range(0, ACH - 15, 16)) + [ACH - 16])):
        dref[pl.ds(o, 16)] = dstage[pl.ds(j * ACH + o, 16)]

    def g_desc(j, rbuf, sem):
      return pltpu.make_async_copy(
          hw_hbm.at[sstage.at[pl.ds(j * ACH, ACH)]], rbuf, sem)

    def s_desc(rbuf, dref, sem):
      return pltpu.make_async_copy(rbuf, acc.at[dref], sem)

    def sblock(sb, _):
      off = pl.multiple_of(base + sb * (ASB * ACH), 8)
      pltpu.sync_copy(src_hbm.at[pl.ds(off, ASB * ACH)], sstage)
      pltpu.sync_copy(dst_hbm.at[pl.ds(off, ASB * ACH)], dstage)
      vcopy(didx0, 0)
      g_desc(0, rows0, sg0).start()

      def pair(t, _):
        j0 = 2 * t
        j1 = j0 + 1
        g_desc(j1, rows1, sg1).start()
        g_desc(j0, rows0, sg0).wait()

        @pl.when(t > 0)
        def _():
          s_desc(rows1, didx1, ss1).wait()

        vcopy(didx1, j1)
        pltpu.async_copy(rows0, acc.at[didx0], ss0, add=True)
        g_desc(j1, rows1, sg1).wait()
        s_desc(rows0, didx0, ss0).wait()
        vcopy(didx0, j0 + 2)
        g_desc(j0 + 2, rows0, sg0).start()
        pltpu.async_copy(rows1, acc.at[didx1], ss1, add=True)
        return _

      lax.fori_loop(0, ASB // 2, pair, None)
      g_desc(ASB - 1, rows0, sg0).wait()
      s_desc(rows1, didx1, ss1).wait()
      pltpu.async_copy(rows0, acc.at[didx0], ss0, add=True)
      s_desc(rows0, didx0, ss0).wait()
      return _

    lax.fori_loop(0, ANSB, sblock, None)
    plsc.subcore_barrier()
    pltpu.sync_copy(acc.at[pl.ds(r0, RPT)],
                    out_hbm.at[cid].at[pl.ds(r0, RPT)])

  return pl.kernel(
      body,
      out_type=jax.ShapeDtypeStruct((NC, NPAD, F), jnp.float32),
      mesh=_MESH,
      scratch_types=[
          pltpu.VMEM((ASB * ACH,), jnp.int32),
          pltpu.VMEM((ASB * ACH,), jnp.int32),
          pltpu.VMEM((ACH,), jnp.int32),
          pltpu.VMEM((ACH,), jnp.int32),
          pltpu.VMEM((ACH, F), jnp.float32),
          pltpu.VMEM((ACH, F), jnp.float32),
          pltpu.VMEM_SHARED((NPAD, F), jnp.float32),
          pltpu.SemaphoreType.DMA,
          pltpu.SemaphoreType.DMA,
          pltpu.SemaphoreType.DMA,
          pltpu.SemaphoreType.DMA,
      ],
  )


_agg_h = _make_agg(H)


def _norms_body(dsrc_ref, ddst_ref, ns_ref, nd_ref):
  ns_ref[...] = lax.rsqrt(jnp.maximum(dsrc_ref[...].astype(jnp.float32), 1.0))
  nd_ref[...] = lax.rsqrt(jnp.maximum(ddst_ref[...].astype(jnp.float32), 1.0))


_norms_tc = pl.pallas_call(
    _norms_body,
    out_shape=[jax.ShapeDtypeStruct((NPAD,), jnp.float32),
               jax.ShapeDtypeStruct((NPAD,), jnp.float32)],
)


def _l1mid_body(c0_ref, c1_ref, nd_ref, b_ref, ns_ref, w_ref, out_ref):
  c = c0_ref[...] + c1_ref[...]
  w1s = w_ref[...] * ns_ref[0:H]
  agg = jnp.dot(c, w1s, preferred_element_type=jnp.float32)
  y = jnp.maximum(agg * nd_ref[...] + b_ref[...], 0.0)
  out_ref[...] = jnp.dot(y * ns_ref[...], w_ref[...],
                         preferred_element_type=jnp.float32)


_l1mid_tc = pl.pallas_call(
    _l1mid_body,
    out_shape=jax.ShapeDtypeStruct((NPAD, H), jnp.float32),
)


def _make_mid(K):
  def body(agg_ref, nd_ref, b_ref, ns_ref, w_ref, out_ref):
    a = agg_ref[0] + agg_ref[1]
    y = jnp.maximum(a * nd_ref[...] + b_ref[...], 0.0)
    out_ref[...] = jnp.dot(y * ns_ref[...], w_ref[...],
                           preferred_element_type=jnp.float32)

  return pl.pallas_call(
      body, out_shape=jax.ShapeDtypeStruct((NPAD, K), jnp.float32))


_mid_tc_h = _make_mid(H)


def _final_body(agg_ref, nd_ref, b_ref, out_ref):
  a = agg_ref[0, :, 0:OUT] + agg_ref[1, :, 0:OUT]
  out_ref[...] = a * nd_ref[...] + b_ref[...]


_final_tc = pl.pallas_call(
    _final_body,
    out_shape=jax.ShapeDtypeStruct((NPAD, OUT), jnp.float32),
)


def kernel(edge_index, W1, b1, W2, b2):
  src = edge_index[0].astype(jnp.int32)
  dst = edge_index[1].astype(jnp.int32)
  zeros_nh = jnp.zeros((NPAD, H), jnp.float32)
  zeros_c = jnp.zeros((CW,), jnp.float32)
  W2p = jnp.pad(W2, ((0, 0), (0, H - OUT)))

  sd = jnp.concatenate([src, dst])
  degs, csum = _prep_kernel(sd, zeros_c)
  dsrc = degs[:NPAD]
  ddst = degs[NPAD:]
  c0 = csum[:NPAD * H].reshape(NPAD, H)
  c1 = csum[NPAD * H:].reshape(NPAD, H)
  ns, nd = _norms_tc(dsrc, ddst)
  ns2 = ns.reshape(NPAD, 1)
  nd2 = nd.reshape(NPAD, 1)
  b1r = b1.reshape(1, H)
  b2r = b2.reshape(1, OUT)

  hw = _l1mid_tc(c0, c1, nd2, b1r, ns2, W1)
  agg = _agg_h(hw, src, dst, zeros_nh)
  hw = _mid_tc_h(agg, nd2, b1r, ns2, W1)
  agg = _agg_h(hw, src, dst, zeros_nh)
  hw2 = _mid_tc_h(agg, nd2, b1r, ns2, W2p)
  agg2 = _agg_h(hw2, src, dst, zeros_nh)
  out = _final_tc(agg2, nd2, b2r)
  return out[:N]


# speedup vs baseline: 9.0055x; 1.1064x over previous
import functools

import jax
import jax.numpy as jnp
from jax import lax
from jax.experimental import pallas as pl
from jax.experimental.pallas import tpu as pltpu
from jax.experimental.pallas import tpu_sc as plsc

N = 10000
E = 320000
H = 128
OUT = 64
NPAD = 10240

NC = 2
NS = 16
NW = NC * NS
RPT = NPAD // NS

EPW = E // NW
ACH = 80
ANCH = EPW // ACH
ASB = 5
ANSB = ANCH // ASB

EPT = E // NS
DCH = 80
DBLK = 2000
CW = NPAD * H + 2048
CPT = NPAD * H // NS

_MESH = plsc.VectorSubcoreMesh(core_axis_name="c", subcore_axis_name="s")


def _prep_body(sd_hbm, zc_hbm, degs_hbm, csum_hbm,
               dstage_v, hist_v, part_v, cstage_s, cstage_d, ci0, ones_v,
               hstage, cacc, sem):
  cid = lax.axis_index("c")
  sid = lax.axis_index("s")
  ones_v[...] = jnp.ones((16,), jnp.float32)

  r0 = pl.multiple_of(sid * RPT, 8)
  z0 = pl.multiple_of(sid * (CW // NS), 8)
  pltpu.sync_copy(zc_hbm.at[pl.ds(z0, CW // NS)], cacc.at[pl.ds(z0, CW // NS)])

  doff = pl.multiple_of(cid * E + sid * EPT, 8)
  wid = cid * NS + sid
  coff = pl.multiple_of(wid * EPW, 8)

  def hzero(i, _):
    hist_v[pl.ds(i * 16, 16)] = jnp.zeros((16,), jnp.int32)
    return _

  lax.fori_loop(0, NPAD // 16, hzero, None)

  def dblk(blk, _):
    off = pl.multiple_of(doff + blk * DBLK, 8)
    pltpu.sync_copy(sd_hbm.at[pl.ds(off, DBLK)], dstage_v)

    def dgrp(g, _):
      idx = dstage_v[pl.ds(g * 16, 16)]
      cnt, last = plsc.scan_count(idx)
      plsc.addupdate_scatter(hist_v, [idx], cnt, mask=last)
      return _

    lax.fori_loop(0, DBLK // 16, dgrp, None)
    return _

  lax.fori_loop(0, EPT // DBLK, dblk, None)

  pltpu.sync_copy(hist_v, hstage.at[sid])
  plsc.subcore_barrier()
  pltpu.sync_copy(hstage.at[0].at[pl.ds(r0, RPT)], hist_v.at[pl.ds(0, RPT)])
  for b in range(1, NS):
    pltpu.sync_copy(hstage.at[b].at[pl.ds(r0, RPT)], part_v)
    for g in range(RPT // 16):
      sl = pl.ds(g * 16, 16)
      hist_v[sl] = hist_v[sl] + part_v[sl]
  pltpu.sync_copy(hist_v.at[pl.ds(0, RPT)],
                  degs_hbm.at[pl.ds(pl.multiple_of(cid * NPAD + r0, 8), RPT)])

  def cblk(blk, _):
    soff = pl.multiple_of(coff + blk * DBLK, 8)
    pltpu.sync_copy(sd_hbm.at[pl.ds(soff, DBLK)], cstage_s)
    pltpu.sync_copy(sd_hbm.at[pl.ds(soff + E, DBLK)], cstage_d)

    def cgrp(g, _):
      sl = pl.ds(g * 16, 16)
      s = cstage_s[sl]
      minv = lax.reduce_min(s, (0,))

      @pl.when(minv < H)
      def _():
        d = cstage_d[sl]
        ci0[...] = jnp.where(s < H, (d << 7) + s, (NPAD * H) + (s & 127))
        pltpu.sync_copy(ones_v, cacc.at[ci0], add=True)
      return _

    lax.fori_loop(0, DBLK // 16, cgrp, None)
    return _

  lax.fori_loop(0, EPW // DBLK, cblk, None)
  plsc.subcore_barrier()

  cd0 = pl.multiple_of(sid * CPT, 8)
  pltpu.sync_copy(
      cacc.at[pl.ds(cd0, CPT)],
      csum_hbm.at[pl.ds(pl.multiple_of(cid * (NPAD * H) + cd0, 8), CPT)])


_prep_kernel = functools.partial(
    pl.kernel,
    out_type=[jax.ShapeDtypeStruct((2 * NPAD,), jnp.int32),
              jax.ShapeDtypeStruct((2 * NPAD * H,), jnp.float32)],
    mesh=_MESH,
    compiler_params=pltpu.CompilerParams(needs_layout_passes=False),
    scratch_types=[
        pltpu.VMEM((DBLK,), jnp.int32),
        pltpu.VMEM((NPAD,), jnp.int32),
        pltpu.VMEM((RPT,), jnp.int32),
        pltpu.VMEM((DBLK,), jnp.int32),
        pltpu.VMEM((DBLK,), jnp.int32),
        pltpu.VMEM((16,), jnp.int32),
        pltpu.VMEM((16,), jnp.float32),
        pltpu.VMEM_SHARED((NS, NPAD), jnp.int32),
        pltpu.VMEM_SHARED((CW,), jnp.float32),
        pltpu.SemaphoreType.DMA,
    ],
)(_prep_body)


def _make_agg(F):
  def body(hw_hbm, src_hbm, dst_hbm, zeros_hbm, out_hbm,
           sstage, dstage, didx0, didx1, rows0, rows1, acc,
           sg0, sg1, ss0, ss1):
    cid = lax.axis_index("c")
    sid = lax.axis_index("s")
    wid = cid * NS + sid
    r0 = pl.multiple_of(sid * RPT, 8)
    pltpu.sync_copy(zeros_hbm.at[pl.ds(r0, RPT)], acc.at[pl.ds(r0, RPT)])
    plsc.subcore_barrier()
    base = wid * EPW

    def vcopy(dref, j):
      for o in sorted(set(list(range(0, ACH - 15, 16)) + [ACH - 16])):
        dref[pl.ds(o, 16)] = dstage[pl.ds(j * ACH + o, 16)]

    def g_desc(j, rbuf, sem):
      return pltpu.make_async_copy(
          hw_hbm.at[sstage.at[pl.ds(j * ACH, ACH)]], rbuf, sem)

    def s_desc(rbuf, dref, sem):
      return pltpu.make_async_copy(rbuf, acc.at[dref], sem)

    def sblock(sb, _):
      off = pl.multiple_of(base + sb * (ASB * ACH), 8)
      pltpu.sync_copy(src_hbm.at[pl.ds(off, ASB * ACH)], sstage)
      pltpu.sync_copy(dst_hbm.at[pl.ds(off, ASB * ACH)], dstage)
      vcopy(didx0, 0)
      g_desc(0, rows0, sg0).start()

      def pair(t, _):
        j0 = 2 * t
        j1 = j0 + 1
        g_desc(j1, rows1, sg1).start()
        g_desc(j0, rows0, sg0).wait()

        @pl.when(t > 0)
        def _():
          s_desc(rows1, didx1, ss1).wait()

        vcopy(didx1, j1)
        pltpu.async_copy(rows0, acc.at[didx0], ss0, add=True)
        g_desc(j1, rows1, sg1).wait()
        s_desc(rows0, didx0, ss0).wait()
        vcopy(didx0, j0 + 2)
        g_desc(j0 + 2, rows0, sg0).start()
        pltpu.async_copy(rows1, acc.at[didx1], ss1, add=True)
        return _

      lax.fori_loop(0, ASB // 2, pair, None)
      g_desc(ASB - 1, rows0, sg0).wait()
      s_desc(rows1, didx1, ss1).wait()
      pltpu.async_copy(rows0, acc.at[didx0], ss0, add=True)
      s_desc(rows0, didx0, ss0).wait()
      return _

    lax.fori_loop(0, ANSB, sblock, None)
    plsc.subcore_barrier()
    pltpu.sync_copy(acc.at[pl.ds(r0, RPT)],
                    out_hbm.at[cid].at[pl.ds(r0, RPT)])

  return pl.kernel(
      body,
      out_type=jax.ShapeDtypeStruct((NC, NPAD, F), jnp.float32),
      mesh=_MESH,
      scratch_types=[
          pltpu.VMEM((ASB * ACH,), jnp.int32),
          pltpu.VMEM((ASB * ACH,), jnp.int32),
          pltpu.VMEM((ACH,), jnp.int32),
          pltpu.VMEM((ACH,), jnp.int32),
          pltpu.VMEM((ACH, F), jnp.float32),
          pltpu.VMEM((ACH, F), jnp.float32),
          pltpu.VMEM_SHARED((NPAD, F), jnp.float32),
          pltpu.SemaphoreType.DMA,
          pltpu.SemaphoreType.DMA,
          pltpu.SemaphoreType.DMA,
          pltpu.SemaphoreType.DMA,
      ],
  )


_agg_h = _make_agg(H)


def _l1mid_body(c0_ref, c1_ref, dsrc_ref, ddst_ref, b_ref, w_ref,
                out_ref, ns_ref, nd_ref):
  ns = lax.rsqrt(jnp.maximum(dsrc_ref[...].astype(jnp.float32), 1.0))
  nd = lax.rsqrt(jnp.maximum(ddst_ref[...].astype(jnp.float32), 1.0))
  ns_ref[...] = ns
  nd_ref[...] = nd
  c = c0_ref[...] + c1_ref[...]
  w1s = w_ref[...] * ns[0:H]
  agg = jnp.dot(c, w1s, preferred_element_type=jnp.float32)
  y = jnp.maximum(agg * nd + b_ref[...], 0.0)
  out_ref[...] = jnp.dot(y * ns, w_ref[...],
                         preferred_element_type=jnp.float32)


_l1mid_tc = pl.pallas_call(
    _l1mid_body,
    out_shape=[jax.ShapeDtypeStruct((NPAD, H), jnp.float32),
               jax.ShapeDtypeStruct((NPAD, 1), jnp.float32),
               jax.ShapeDtypeStruct((NPAD, 1), jnp.float32)],
)


def _make_mid(K):
  def body(agg_ref, nd_ref, b_ref, ns_ref, w_ref, out_ref):
    a = agg_ref[0] + agg_ref[1]
    y = jnp.maximum(a * nd_ref[...] + b_ref[...], 0.0)
    out_ref[...] = jnp.dot(y * ns_ref[...], w_ref[...],
                           preferred_element_type=jnp.float32)

  return pl.pallas_call(
      body, out_shape=jax.ShapeDtypeStruct((NPAD, K), jnp.float32))


_mid_tc_h = _make_mid(H)


def _final_body(agg_ref, nd_ref, b_ref, out_ref):
  a = agg_ref[0, :, 0:OUT] + agg_ref[1, :, 0:OUT]
  out_ref[...] = a * nd_ref[...] + b_ref[...]


_final_tc = pl.pallas_call(
    _final_body,
    out_shape=jax.ShapeDtypeStruct((NPAD, OUT), jnp.float32),
)


def kernel(edge_index, W1, b1, W2, b2):
  src = edge_index[0].astype(jnp.int32)
  dst = edge_index[1].astype(jnp.int32)
  zeros_nh = jnp.zeros((NPAD, H), jnp.float32)
  zeros_c = jnp.zeros((CW,), jnp.float32)
  W2p = jnp.pad(W2, ((0, 0), (0, H - OUT)))

  sd = jnp.concatenate([src, dst])
  degs, csum = _prep_kernel(sd, zeros_c)
  dsrc2 = degs[:NPAD].reshape(NPAD, 1)
  ddst2 = degs[NPAD:].reshape(NPAD, 1)
  c0 = csum[:NPAD * H].reshape(NPAD, H)
  c1 = csum[NPAD * H:].reshape(NPAD, H)
  b1r = b1.reshape(1, H)
  b2r = b2.reshape(1, OUT)

  hw, ns2, nd2 = _l1mid_tc(c0, c1, dsrc2, ddst2, b1r, W1)
  agg = _agg_h(hw, src, dst, zeros_nh)
  hw = _mid_tc_h(agg, nd2, b1r, ns2, W1)
  agg = _agg_h(hw, src, dst, zeros_nh)
  hw2 = _mid_tc_h(agg, nd2, b1r, ns2, W2p)
  agg2 = _agg_h(hw2, src, dst, zeros_nh)
  out = _final_tc(agg2, nd2, b2r)
  return out[:N]


# speedup vs baseline: 10.3140x; 1.1453x over previous
import functools

import jax
import jax.numpy as jnp
from jax import lax
from jax.experimental import pallas as pl
from jax.experimental.pallas import tpu as pltpu
from jax.experimental.pallas import tpu_sc as plsc

N = 10000
E = 320000
H = 128
OUT = 64
NPAD = 10240

NC = 2
NS = 16
NW = NC * NS
RPT = NPAD // NS

EPW = E // NW
ACH = 80
ANCH = EPW // ACH
ASB = 25
ANSB = ANCH // ASB

EPT = E // NS
DCH = 80
DBLK = 2000
CW = NPAD * H + 2048
CPT = NPAD * H // NS

_MESH = plsc.VectorSubcoreMesh(core_axis_name="c", subcore_axis_name="s")


def _prep_body(sd_hbm, zc_hbm, degs_hbm, csum_hbm,
               dstage_v, hist_v, part_v, cstage_s, cstage_d, ci0, ones_v,
               hstage, cacc, sem):
  cid = lax.axis_index("c")
  sid = lax.axis_index("s")
  ones_v[...] = jnp.ones((16,), jnp.float32)

  r0 = pl.multiple_of(sid * RPT, 8)
  z0 = pl.multiple_of(sid * (CW // NS), 8)
  pltpu.sync_copy(zc_hbm.at[pl.ds(z0, CW // NS)], cacc.at[pl.ds(z0, CW // NS)])

  doff = pl.multiple_of(cid * E + sid * EPT, 8)
  wid = cid * NS + sid
  coff = pl.multiple_of(wid * EPW, 8)

  def hzero(i, _):
    hist_v[pl.ds(i * 16, 16)] = jnp.zeros((16,), jnp.int32)
    return _

  lax.fori_loop(0, NPAD // 16, hzero, None)

  def dblk(blk, _):
    off = pl.multiple_of(doff + blk * DBLK, 8)
    pltpu.sync_copy(sd_hbm.at[pl.ds(off, DBLK)], dstage_v)

    def dgrp(g, _):
      idx = dstage_v[pl.ds(g * 16, 16)]
      cnt, last = plsc.scan_count(idx)
      plsc.addupdate_scatter(hist_v, [idx], cnt, mask=last)
      return _

    lax.fori_loop(0, DBLK // 16, dgrp, None)
    return _

  lax.fori_loop(0, EPT // DBLK, dblk, None)

  pltpu.sync_copy(hist_v, hstage.at[sid])
  plsc.subcore_barrier()
  pltpu.sync_copy(hstage.at[0].at[pl.ds(r0, RPT)], hist_v.at[pl.ds(0, RPT)])
  for b in range(1, NS):
    pltpu.sync_copy(hstage.at[b].at[pl.ds(r0, RPT)], part_v)
    for g in range(RPT // 16):
      sl = pl.ds(g * 16, 16)
      hist_v[sl] = hist_v[sl] + part_v[sl]
  pltpu.sync_copy(hist_v.at[pl.ds(0, RPT)],
                  degs_hbm.at[pl.ds(pl.multiple_of(cid * NPAD + r0, 8), RPT)])

  def cblk(blk, _):
    soff = pl.multiple_of(coff + blk * DBLK, 8)
    pltpu.sync_copy(sd_hbm.at[pl.ds(soff, DBLK)], cstage_s)
    pltpu.sync_copy(sd_hbm.at[pl.ds(soff + E, DBLK)], cstage_d)

    def cgrp(g, _):
      sl = pl.ds(g * 16, 16)
      s = cstage_s[sl]
      minv = lax.reduce_min(s, (0,))

      @pl.when(minv < H)
      def _():
        d = cstage_d[sl]
        ci0[...] = jnp.where(s < H, (d << 7) + s, (NPAD * H) + (s & 127))
        pltpu.sync_copy(ones_v, cacc.at[ci0], add=True)
      return _

    lax.fori_loop(0, DBLK // 16, cgrp, None)
    return _

  lax.fori_loop(0, EPW // DBLK, cblk, None)
  plsc.subcore_barrier()

  cd0 = pl.multiple_of(sid * CPT, 8)
  pltpu.sync_copy(
      cacc.at[pl.ds(cd0, CPT)],
      csum_hbm.at[pl.ds(pl.multiple_of(cid * (NPAD * H) + cd0, 8), CPT)])


_prep_kernel = functools.partial(
    pl.kernel,
    out_type=[jax.ShapeDtypeStruct((2 * NPAD,), jnp.int32),
              jax.ShapeDtypeStruct((2 * NPAD * H,), jnp.float32)],
    mesh=_MESH,
    compiler_params=pltpu.CompilerParams(needs_layout_passes=False),
    scratch_types=[
        pltpu.VMEM((DBLK,), jnp.int32),
        pltpu.VMEM((NPAD,), jnp.int32),
        pltpu.VMEM((RPT,), jnp.int32),
        pltpu.VMEM((DBLK,), jnp.int32),
        pltpu.VMEM((DBLK,), jnp.int32),
        pltpu.VMEM((16,), jnp.int32),
        pltpu.VMEM((16,), jnp.float32),
        pltpu.VMEM_SHARED((NS, NPAD), jnp.int32),
        pltpu.VMEM_SHARED((CW,), jnp.float32),
        pltpu.SemaphoreType.DMA,
    ],
)(_prep_body)


def _make_agg(F):
  def body(hw_hbm, src_hbm, dst_hbm, zeros_hbm, out_hbm,
           sstage, dstage, didx0, didx1, rows0, rows1, acc,
           sg0, sg1, ss0, ss1):
    cid = lax.axis_index("c")
    sid = lax.axis_index("s")
    wid = cid * NS + sid
    r0 = pl.multiple_of(sid * RPT, 8)
    pltpu.sync_copy(zeros_hbm.at[pl.ds(r0, RPT)], acc.at[pl.ds(r0, RPT)])
    plsc.subcore_barrier()
    base = wid * EPW

    def vcopy(dref, j):
      for o in sorted(set(list(range(0, ACH - 15, 16)) + [ACH - 16])):
        dref[pl.ds(o, 16)] = dstage[pl.ds(j * ACH + o, 16)]

    def g_desc(j, rbuf, sem):
      return pltpu.make_async_copy(
          hw_hbm.at[sstage.at[pl.ds(j * ACH, ACH)]], rbuf, sem)

    def s_desc(rbuf, dref, sem):
      return pltpu.make_async_copy(rbuf, acc.at[dref], sem)

    def sblock(sb, _):
      off = pl.multiple_of(base + sb * (ASB * ACH), 8)
      pltpu.sync_copy(src_hbm.at[pl.ds(off, ASB * ACH)], sstage)
      pltpu.sync_copy(dst_hbm.at[pl.ds(off, ASB * ACH)], dstage)
      vcopy(didx0, 0)
      g_desc(0, rows0, sg0).start()

      def pair(t, _):
        j0 = 2 * t
        j1 = j0 + 1
        g_desc(j1, rows1, sg1).start()
        g_desc(j0, rows0, sg0).wait()

        @pl.when(t > 0)
        def _():
          s_desc(rows1, didx1, ss1).wait()

        vcopy(didx1, j1)
        pltpu.async_copy(rows0, acc.at[didx0], ss0, add=True)
        g_desc(j1, rows1, sg1).wait()
        s_desc(rows0, didx0, ss0).wait()
        vcopy(didx0, j0 + 2)
        g_desc(j0 + 2, rows0, sg0).start()
        pltpu.async_copy(rows1, acc.at[didx1], ss1, add=True)
        return _

      lax.fori_loop(0, ASB // 2, pair, None)
      g_desc(ASB - 1, rows0, sg0).wait()
      s_desc(rows1, didx1, ss1).wait()
      pltpu.async_copy(rows0, acc.at[didx0], ss0, add=True)
      s_desc(rows0, didx0, ss0).wait()
      return _

    lax.fori_loop(0, ANSB, sblock, None)
    plsc.subcore_barrier()
    pltpu.sync_copy(acc.at[pl.ds(r0, RPT)],
                    out_hbm.at[cid].at[pl.ds(r0, RPT)])

  return pl.kernel(
      body,
      out_type=jax.ShapeDtypeStruct((NC, NPAD, F), jnp.float32),
      mesh=_MESH,
      scratch_types=[
          pltpu.VMEM((ASB * ACH,), jnp.int32),
          pltpu.VMEM((ASB * ACH,), jnp.int32),
          pltpu.VMEM((ACH,), jnp.int32),
          pltpu.VMEM((ACH,), jnp.int32),
          pltpu.VMEM((ACH, F), jnp.float32),
          pltpu.VMEM((ACH, F), jnp.float32),
          pltpu.VMEM_SHARED((NPAD, F), jnp.float32),
          pltpu.SemaphoreType.DMA,
          pltpu.SemaphoreType.DMA,
          pltpu.SemaphoreType.DMA,
          pltpu.SemaphoreType.DMA,
      ],
  )


_agg_h = _make_agg(H)


def _l1mid_body(c0_ref, c1_ref, dsrc_ref, ddst_ref, b_ref, w_ref,
                out_ref, ns_ref, nd_ref):
  ns = lax.rsqrt(jnp.maximum(dsrc_ref[...].astype(jnp.float32), 1.0))
  nd = lax.rsqrt(jnp.maximum(ddst_ref[...].astype(jnp.float32), 1.0))
  ns_ref[...] = ns
  nd_ref[...] = nd
  c = c0_ref[...] + c1_ref[...]
  w1s = w_ref[...] * ns[0:H]
  agg = jnp.dot(c, w1s, preferred_element_type=jnp.float32)
  y = jnp.maximum(agg * nd + b_ref[...], 0.0)
  out_ref[...] = jnp.dot(y * ns, w_ref[...],
                         preferred_element_type=jnp.float32)


_l1mid_tc = pl.pallas_call(
    _l1mid_body,
    out_shape=[jax.ShapeDtypeStruct((NPAD, H), jnp.float32),
               jax.ShapeDtypeStruct((NPAD, 1), jnp.float32),
               jax.ShapeDtypeStruct((NPAD, 1), jnp.float32)],
)


def _make_mid(K):
  def body(agg_ref, nd_ref, b_ref, ns_ref, w_ref, out_ref):
    a = agg_ref[0] + agg_ref[1]
    y = jnp.maximum(a * nd_ref[...] + b_ref[...], 0.0)
    out_ref[...] = jnp.dot(y * ns_ref[...], w_ref[...],
                           preferred_element_type=jnp.float32)

  return pl.pallas_call(
      body, out_shape=jax.ShapeDtypeStruct((NPAD, K), jnp.float32))


_mid_tc_h = _make_mid(H)


def _final_body(agg_ref, nd_ref, b_ref, out_ref):
  a = agg_ref[0, :, 0:OUT] + agg_ref[1, :, 0:OUT]
  out_ref[...] = a * nd_ref[...] + b_ref[...]


_final_tc = pl.pallas_call(
    _final_body,
    out_shape=jax.ShapeDtypeStruct((NPAD, OUT), jnp.float32),
)


def kernel(edge_index, W1, b1, W2, b2):
  src = edge_index[0].astype(jnp.int32)
  dst = edge_index[1].astype(jnp.int32)
  zeros_nh = jnp.zeros((NPAD, H), jnp.float32)
  zeros_c = jnp.zeros((CW,), jnp.float32)
  W2p = jnp.pad(W2, ((0, 0), (0, H - OUT)))

  sd = jnp.concatenate([src, dst])
  degs, csum = _prep_kernel(sd, zeros_c)
  dsrc2 = degs[:NPAD].reshape(NPAD, 1)
  ddst2 = degs[NPAD:].reshape(NPAD, 1)
  c0 = csum[:NPAD * H].reshape(NPAD, H)
  c1 = csum[NPAD * H:].reshape(NPAD, H)
  b1r = b1.reshape(1, H)
  b2r = b2.reshape(1, OUT)

  hw, ns2, nd2 = _l1mid_tc(c0, c1, dsrc2, ddst2, b1r, W1)
  agg = _agg_h(hw, src, dst, zeros_nh)
  hw = _mid_tc_h(agg, nd2, b1r, ns2, W1)
  agg = _agg_h(hw, src, dst, zeros_nh)
  hw2 = _mid_tc_h(agg, nd2, b1r, ns2, W2p)
  agg2 = _agg_h(hw2, src, dst, zeros_nh)
  out = _final_tc(agg2, nd2, b2r)
  return out[:N]


# speedup vs baseline: 10.6804x; 1.0355x over previous
import functools

import jax
import jax.numpy as jnp
from jax import lax
from jax.experimental import pallas as pl
from jax.experimental.pallas import tpu as pltpu
from jax.experimental.pallas import tpu_sc as plsc

N = 10000
E = 320000
H = 128
OUT = 64
NPAD = 10240

NC = 2
NS = 16
NW = NC * NS
RPT = NPAD // NS

EPW = E // NW
ACH = 80
ANCH = EPW // ACH
ASB = 125
ANSB = ANCH // ASB

EPT = E // NS
DCH = 80
DBLK = 2000
CW = NPAD * H + 2048
CPT = NPAD * H // NS

_MESH = plsc.VectorSubcoreMesh(core_axis_name="c", subcore_axis_name="s")


def _prep_body(sd_hbm, zc_hbm, degs_hbm, csum_hbm,
               dstage_v, hist_v, part_v, cstage_s, cstage_d, ci0, ones_v,
               hstage, cacc, sem):
  cid = lax.axis_index("c")
  sid = lax.axis_index("s")
  ones_v[...] = jnp.ones((16,), jnp.float32)

  r0 = pl.multiple_of(sid * RPT, 8)
  z0 = pl.multiple_of(sid * (CW // NS), 8)
  pltpu.sync_copy(zc_hbm.at[pl.ds(z0, CW // NS)], cacc.at[pl.ds(z0, CW // NS)])

  doff = pl.multiple_of(cid * E + sid * EPT, 8)
  wid = cid * NS + sid
  coff = pl.multiple_of(wid * EPW, 8)

  def hzero(i, _):
    hist_v[pl.ds(i * 16, 16)] = jnp.zeros((16,), jnp.int32)
    return _

  lax.fori_loop(0, NPAD // 16, hzero, None)

  def dblk(blk, _):
    off = pl.multiple_of(doff + blk * DBLK, 8)
    pltpu.sync_copy(sd_hbm.at[pl.ds(off, DBLK)], dstage_v)

    def dgrp(g, _):
      idx = dstage_v[pl.ds(g * 16, 16)]
      cnt, last = plsc.scan_count(idx)
      plsc.addupdate_scatter(hist_v, [idx], cnt, mask=last)
      return _

    lax.fori_loop(0, DBLK // 16, dgrp, None)
    return _

  lax.fori_loop(0, EPT // DBLK, dblk, None)

  pltpu.sync_copy(hist_v, hstage.at[sid])
  plsc.subcore_barrier()
  pltpu.sync_copy(hstage.at[0].at[pl.ds(r0, RPT)], hist_v.at[pl.ds(0, RPT)])
  for b in range(1, NS):
    pltpu.sync_copy(hstage.at[b].at[pl.ds(r0, RPT)], part_v)
    for g in range(RPT // 16):
      sl = pl.ds(g * 16, 16)
      hist_v[sl] = hist_v[sl] + part_v[sl]
  pltpu.sync_copy(hist_v.at[pl.ds(0, RPT)],
                  degs_hbm.at[pl.ds(pl.multiple_of(cid * NPAD + r0, 8), RPT)])

  def cblk(blk, _):
    soff = pl.multiple_of(coff + blk * DBLK, 8)
    pltpu.sync_copy(sd_hbm.at[pl.ds(soff, DBLK)], cstage_s)
    pltpu.sync_copy(sd_hbm.at[pl.ds(soff + E, DBLK)], cstage_d)

    def cgrp(g, _):
      sl = pl.ds(g * 16, 16)
      s = cstage_s[sl]
      minv = lax.reduce_min(s, (0,))

      @pl.when(minv < H)
      def _():
        d = cstage_d[sl]
        ci0[...] = jnp.where(s < H, (d << 7) + s, (NPAD * H) + (s & 127))
        pltpu.sync_copy(ones_v, cacc.at[ci0], add=True)
      return _

    lax.fori_loop(0, DBLK // 16, cgrp, None)
    return _

  lax.fori_loop(0, EPW // DBLK, cblk, None)
  plsc.subcore_barrier()

  cd0 = pl.multiple_of(sid * CPT, 8)
  pltpu.sync_copy(
      cacc.at[pl.ds(cd0, CPT)],
      csum_hbm.at[pl.ds(pl.multiple_of(cid * (NPAD * H) + cd0, 8), CPT)])


_prep_kernel = functools.partial(
    pl.kernel,
    out_type=[jax.ShapeDtypeStruct((2 * NPAD,), jnp.int32),
              jax.ShapeDtypeStruct((2 * NPAD * H,), jnp.float32)],
    mesh=_MESH,
    compiler_params=pltpu.CompilerParams(needs_layout_passes=False),
    scratch_types=[
        pltpu.VMEM((DBLK,), jnp.int32),
        pltpu.VMEM((NPAD,), jnp.int32),
        pltpu.VMEM((RPT,), jnp.int32),
        pltpu.VMEM((DBLK,), jnp.int32),
        pltpu.VMEM((DBLK,), jnp.int32),
        pltpu.VMEM((16,), jnp.int32),
        pltpu.VMEM((16,), jnp.float32),
        pltpu.VMEM_SHARED((NS, NPAD), jnp.int32),
        pltpu.VMEM_SHARED((CW,), jnp.float32),
        pltpu.SemaphoreType.DMA,
    ],
)(_prep_body)


def _make_agg(F):
  def body(hw_hbm, src_hbm, dst_hbm, zeros_hbm, out_hbm,
           sstage, dstage, didx0, didx1, rows0, rows1, acc,
           sg0, sg1, ss0, ss1):
    cid = lax.axis_index("c")
    sid = lax.axis_index("s")
    wid = cid * NS + sid
    r0 = pl.multiple_of(sid * RPT, 8)
    pltpu.sync_copy(zeros_hbm.at[pl.ds(r0, RPT)], acc.at[pl.ds(r0, RPT)])
    plsc.subcore_barrier()
    base = wid * EPW

    def vcopy(dref, j):
      for o in sorted(set(list(range(0, ACH - 15, 16)) + [ACH - 16])):
        dref[pl.ds(o, 16)] = dstage[pl.ds(j * ACH + o, 16)]

    def g_desc(j, rbuf, sem):
      return pltpu.make_async_copy(
          hw_hbm.at[sstage.at[pl.ds(j * ACH, ACH)]], rbuf, sem)

    def s_desc(rbuf, dref, sem):
      return pltpu.make_async_copy(rbuf, acc.at[dref], sem)

    def sblock(sb, _):
      off = pl.multiple_of(base + sb * (ASB * ACH), 8)
      pltpu.sync_copy(src_hbm.at[pl.ds(off, ASB * ACH)], sstage)
      pltpu.sync_copy(dst_hbm.at[pl.ds(off, ASB * ACH)], dstage)
      vcopy(didx0, 0)
      g_desc(0, rows0, sg0).start()

      def pair(t, _):
        j0 = 2 * t
        j1 = j0 + 1
        g_desc(j1, rows1, sg1).start()
        g_desc(j0, rows0, sg0).wait()

        @pl.when(t > 0)
        def _():
          s_desc(rows1, didx1, ss1).wait()

        vcopy(didx1, j1)
        pltpu.async_copy(rows0, acc.at[didx0], ss0, add=True)
        g_desc(j1, rows1, sg1).wait()
        s_desc(rows0, didx0, ss0).wait()
        vcopy(didx0, j0 + 2)
        g_desc(j0 + 2, rows0, sg0).start()
        pltpu.async_copy(rows1, acc.at[didx1], ss1, add=True)
        return _

      lax.fori_loop(0, ASB // 2, pair, None)
      g_desc(ASB - 1, rows0, sg0).wait()
      s_desc(rows1, didx1, ss1).wait()
      pltpu.async_copy(rows0, acc.at[didx0], ss0, add=True)
      s_desc(rows0, didx0, ss0).wait()
      return _

    lax.fori_loop(0, ANSB, sblock, None)
    plsc.subcore_barrier()
    pltpu.sync_copy(acc.at[pl.ds(r0, RPT)],
                    out_hbm.at[cid].at[pl.ds(r0, RPT)])

  return pl.kernel(
      body,
      out_type=jax.ShapeDtypeStruct((NC, NPAD, F), jnp.float32),
      mesh=_MESH,
      scratch_types=[
          pltpu.VMEM((ASB * ACH,), jnp.int32),
          pltpu.VMEM((ASB * ACH,), jnp.int32),
          pltpu.VMEM((ACH,), jnp.int32),
          pltpu.VMEM((ACH,), jnp.int32),
          pltpu.VMEM((ACH, F), jnp.float32),
          pltpu.VMEM((ACH, F), jnp.float32),
          pltpu.VMEM_SHARED((NPAD, F), jnp.float32),
          pltpu.SemaphoreType.DMA,
          pltpu.SemaphoreType.DMA,
          pltpu.SemaphoreType.DMA,
          pltpu.SemaphoreType.DMA,
      ],
  )


_agg_h = _make_agg(H)


def _l1mid_body(c0_ref, c1_ref, dsrc_ref, ddst_ref, b_ref, w_ref,
                out_ref, ns_ref, nd_ref):
  ns = lax.rsqrt(jnp.maximum(dsrc_ref[...].astype(jnp.float32), 1.0))
  nd = lax.rsqrt(jnp.maximum(ddst_ref[...].astype(jnp.float32), 1.0))
  ns_ref[...] = ns
  nd_ref[...] = nd
  c = c0_ref[...] + c1_ref[...]
  w1s = w_ref[...] * ns[0:H]
  agg = jnp.dot(c, w1s, preferred_element_type=jnp.float32)
  y = jnp.maximum(agg * nd + b_ref[...], 0.0)
  out_ref[...] = jnp.dot(y * ns, w_ref[...],
                         preferred_element_type=jnp.float32)


_l1mid_tc = pl.pallas_call(
    _l1mid_body,
    out_shape=[jax.ShapeDtypeStruct((NPAD, H), jnp.float32),
               jax.ShapeDtypeStruct((NPAD, 1), jnp.float32),
               jax.ShapeDtypeStruct((NPAD, 1), jnp.float32)],
)


def _make_mid(K):
  def body(agg_ref, nd_ref, b_ref, ns_ref, w_ref, out_ref):
    a = agg_ref[0] + agg_ref[1]
    y = jnp.maximum(a * nd_ref[...] + b_ref[...], 0.0)
    out_ref[...] = jnp.dot(y * ns_ref[...], w_ref[...],
                           preferred_element_type=jnp.float32)

  return pl.pallas_call(
      body, out_shape=jax.ShapeDtypeStruct((NPAD, K), jnp.float32))


_mid_tc_h = _make_mid(H)


def _final_body(agg_ref, nd_ref, b_ref, out_ref):
  a = agg_ref[0, :, 0:OUT] + agg_ref[1, :, 0:OUT]
  out_ref[...] = a * nd_ref[...] + b_ref[...]


_final_tc = pl.pallas_call(
    _final_body,
    out_shape=jax.ShapeDtypeStruct((NPAD, OUT), jnp.float32),
)


def kernel(edge_index, W1, b1, W2, b2):
  src = edge_index[0].astype(jnp.int32)
  dst = edge_index[1].astype(jnp.int32)
  zeros_nh = jnp.zeros((NPAD, H), jnp.float32)
  zeros_c = jnp.zeros((CW,), jnp.float32)
  W2p = jnp.pad(W2, ((0, 0), (0, H - OUT)))

  sd = jnp.concatenate([src, dst])
  degs, csum = _prep_kernel(sd, zeros_c)
  dsrc2 = degs[:NPAD].reshape(NPAD, 1)
  ddst2 = degs[NPAD:].reshape(NPAD, 1)
  c0 = csum[:NPAD * H].reshape(NPAD, H)
  c1 = csum[NPAD * H:].reshape(NPAD, H)
  b1r = b1.reshape(1, H)
  b2r = b2.reshape(1, OUT)

  hw, ns2, nd2 = _l1mid_tc(c0, c1, dsrc2, ddst2, b1r, W1)
  agg = _agg_h(hw, src, dst, zeros_nh)
  hw = _mid_tc_h(agg, nd2, b1r, ns2, W1)
  agg = _agg_h(hw, src, dst, zeros_nh)
  hw2 = _mid_tc_h(agg, nd2, b1r, ns2, W2p)
  agg2 = _agg_h(hw2, src, dst, zeros_nh)
  out = _final_tc(agg2, nd2, b2r)
  return out[:N]


# speedup vs baseline: 10.6867x; 1.0006x over previous
import functools

import jax
import jax.numpy as jnp
from jax import lax
from jax.experimental import pallas as pl
from jax.experimental.pallas import tpu as pltpu
from jax.experimental.pallas import tpu_sc as plsc

N = 10000
E = 320000
H = 128
OUT = 64
NPAD = 10240

NC = 2
NS = 16
NW = NC * NS
RPT = NPAD // NS

EPW = E // NW
ACH = 80
ANCH = EPW // ACH
ASB = 125
ANSB = ANCH // ASB

EPT = E // NS
DCH = 80
DBLK = 2000
CW = NPAD * H + 2048
CPT = NPAD * H // NS

_MESH = plsc.VectorSubcoreMesh(core_axis_name="c", subcore_axis_name="s")


def _prep_body(sd_hbm, zc_hbm, degs_hbm, csum_hbm,
               dstage_v, hist_v, part_v, cstage_s, cstage_d, ci0, ones_v,
               hstage, cacc, sem):
  cid = lax.axis_index("c")
  sid = lax.axis_index("s")
  ones_v[...] = jnp.ones((16,), jnp.float32)

  r0 = pl.multiple_of(sid * RPT, 8)
  z0 = pl.multiple_of(sid * (CW // NS), 8)
  pltpu.sync_copy(zc_hbm.at[pl.ds(z0, CW // NS)], cacc.at[pl.ds(z0, CW // NS)])

  doff = pl.multiple_of(cid * E + sid * EPT, 8)
  wid = cid * NS + sid
  coff = pl.multiple_of(wid * EPW, 8)

  def hzero(i, _):
    hist_v[pl.ds(i * 16, 16)] = jnp.zeros((16,), jnp.int32)
    return _

  lax.fori_loop(0, NPAD // 16, hzero, None)

  def dblk(blk, _):
    off = pl.multiple_of(doff + blk * DBLK, 8)
    pltpu.sync_copy(sd_hbm.at[pl.ds(off, DBLK)], dstage_v)

    def dgrp(g, _):
      for k in range(5):
        idx = dstage_v[pl.ds((g * 5 + k) * 16, 16)]
        cnt, last = plsc.scan_count(idx)
        plsc.addupdate_scatter(hist_v, [idx], cnt, mask=last)
      return _

    lax.fori_loop(0, DBLK // 80, dgrp, None)
    return _

  lax.fori_loop(0, EPT // DBLK, dblk, None)

  pltpu.sync_copy(hist_v, hstage.at[sid])
  plsc.subcore_barrier()
  pltpu.sync_copy(hstage.at[0].at[pl.ds(r0, RPT)], hist_v.at[pl.ds(0, RPT)])
  for b in range(1, NS):
    pltpu.sync_copy(hstage.at[b].at[pl.ds(r0, RPT)], part_v)
    for g in range(RPT // 16):
      sl = pl.ds(g * 16, 16)
      hist_v[sl] = hist_v[sl] + part_v[sl]
  pltpu.sync_copy(hist_v.at[pl.ds(0, RPT)],
                  degs_hbm.at[pl.ds(pl.multiple_of(cid * NPAD + r0, 8), RPT)])

  def cblk(blk, _):
    soff = pl.multiple_of(coff + blk * DBLK, 8)
    pltpu.sync_copy(sd_hbm.at[pl.ds(soff, DBLK)], cstage_s)
    pltpu.sync_copy(sd_hbm.at[pl.ds(soff + E, DBLK)], cstage_d)

    def cgrp(g, _):
      sl = pl.ds(g * 16, 16)
      s = cstage_s[sl]
      minv = lax.reduce_min(s, (0,))

      @pl.when(minv < H)
      def _():
        d = cstage_d[sl]
        ci0[...] = jnp.where(s < H, (d << 7) + s, (NPAD * H) + (s & 127))
        pltpu.sync_copy(ones_v, cacc.at[ci0], add=True)
      return _

    lax.fori_loop(0, DBLK // 16, cgrp, None)
    return _

  lax.fori_loop(0, EPW // DBLK, cblk, None)
  plsc.subcore_barrier()

  cd0 = pl.multiple_of(sid * CPT, 8)
  pltpu.sync_copy(
      cacc.at[pl.ds(cd0, CPT)],
      csum_hbm.at[pl.ds(pl.multiple_of(cid * (NPAD * H) + cd0, 8), CPT)])


_prep_kernel = functools.partial(
    pl.kernel,
    out_type=[jax.ShapeDtypeStruct((2 * NPAD,), jnp.int32),
              jax.ShapeDtypeStruct((2 * NPAD * H,), jnp.float32)],
    mesh=_MESH,
    compiler_params=pltpu.CompilerParams(needs_layout_passes=False),
    scratch_types=[
        pltpu.VMEM((DBLK,), jnp.int32),
        pltpu.VMEM((NPAD,), jnp.int32),
        pltpu.VMEM((RPT,), jnp.int32),
        pltpu.VMEM((DBLK,), jnp.int32),
        pltpu.VMEM((DBLK,), jnp.int32),
        pltpu.VMEM((16,), jnp.int32),
        pltpu.VMEM((16,), jnp.float32),
        pltpu.VMEM_SHARED((NS, NPAD), jnp.int32),
        pltpu.VMEM_SHARED((CW,), jnp.float32),
        pltpu.SemaphoreType.DMA,
    ],
)(_prep_body)


def _make_agg(F):
  def body(hw_hbm, src_hbm, dst_hbm, zeros_hbm, out_hbm,
           sstage, dstage, didx0, didx1, rows0, rows1, acc,
           sg0, sg1, ss0, ss1):
    cid = lax.axis_index("c")
    sid = lax.axis_index("s")
    wid = cid * NS + sid
    r0 = pl.multiple_of(sid * RPT, 8)
    pltpu.sync_copy(zeros_hbm.at[pl.ds(r0, RPT)], acc.at[pl.ds(r0, RPT)])
    plsc.subcore_barrier()
    base = wid * EPW

    def vcopy(dref, j):
      for o in sorted(set(list(range(0, ACH - 15, 16)) + [ACH - 16])):
        dref[pl.ds(o, 16)] = dstage[pl.ds(j * ACH + o, 16)]

    def g_desc(j, rbuf, sem):
      return pltpu.make_async_copy(
          hw_hbm.at[sstage.at[pl.ds(j * ACH, ACH)]], rbuf, sem)

    def s_desc(rbuf, dref, sem):
      return pltpu.make_async_copy(rbuf, acc.at[dref], sem)

    def sblock(sb, _):
      off = pl.multiple_of(base + sb * (ASB * ACH), 8)
      pltpu.sync_copy(src_hbm.at[pl.ds(off, ASB * ACH)], sstage)
      pltpu.sync_copy(dst_hbm.at[pl.ds(off, ASB * ACH)], dstage)
      vcopy(didx0, 0)
      g_desc(0, rows0, sg0).start()

      def pair(t, _):
        j0 = 2 * t
        j1 = j0 + 1
        g_desc(j1, rows1, sg1).start()
        g_desc(j0, rows0, sg0).wait()

        @pl.when(t > 0)
        def _():
          s_desc(rows1, didx1, ss1).wait()

        vcopy(didx1, j1)
        pltpu.async_copy(rows0, acc.at[didx0], ss0, add=True)
        g_desc(j1, rows1, sg1).wait()
        s_desc(rows0, didx0, ss0).wait()
        vcopy(didx0, j0 + 2)
        g_desc(j0 + 2, rows0, sg0).start()
        pltpu.async_copy(rows1, acc.at[didx1], ss1, add=True)
        return _

      lax.fori_loop(0, ASB // 2, pair, None)
      g_desc(ASB - 1, rows0, sg0).wait()
      s_desc(rows1, didx1, ss1).wait()
      pltpu.async_copy(rows0, acc.at[didx0], ss0, add=True)
      s_desc(rows0, didx0, ss0).wait()
      return _

    lax.fori_loop(0, ANSB, sblock, None)
    plsc.subcore_barrier()
    pltpu.sync_copy(acc.at[pl.ds(r0, RPT)],
                    out_hbm.at[cid].at[pl.ds(r0, RPT)])

  return pl.kernel(
      body,
      out_type=jax.ShapeDtypeStruct((NC, NPAD, F), jnp.float32),
      mesh=_MESH,
      scratch_types=[
          pltpu.VMEM((ASB * ACH,), jnp.int32),
          pltpu.VMEM((ASB * ACH,), jnp.int32),
          pltpu.VMEM((ACH,), jnp.int32),
          pltpu.VMEM((ACH,), jnp.int32),
          pltpu.VMEM((ACH, F), jnp.float32),
          pltpu.VMEM((ACH, F), jnp.float32),
          pltpu.VMEM_SHARED((NPAD, F), jnp.float32),
          pltpu.SemaphoreType.DMA,
          pltpu.SemaphoreType.DMA,
          pltpu.SemaphoreType.DMA,
          pltpu.SemaphoreType.DMA,
      ],
  )


_agg_h = _make_agg(H)


def _l1mid_body(c0_ref, c1_ref, dsrc_ref, ddst_ref, b_ref, w_ref,
                out_ref, ns_ref, nd_ref):
  ns = lax.rsqrt(jnp.maximum(dsrc_ref[...].astype(jnp.float32), 1.0))
  nd = lax.rsqrt(jnp.maximum(ddst_ref[...].astype(jnp.float32), 1.0))
  ns_ref[...] = ns
  nd_ref[...] = nd
  c = c0_ref[...] + c1_ref[...]
  w1s = w_ref[...] * ns[0:H]
  agg = jnp.dot(c, w1s, preferred_element_type=jnp.float32)
  y = jnp.maximum(agg * nd + b_ref[...], 0.0)
  out_ref[...] = jnp.dot(y * ns, w_ref[...],
                         preferred_element_type=jnp.float32)


_l1mid_tc = pl.pallas_call(
    _l1mid_body,
    out_shape=[jax.ShapeDtypeStruct((NPAD, H), jnp.float32),
               jax.ShapeDtypeStruct((NPAD, 1), jnp.float32),
               jax.ShapeDtypeStruct((NPAD, 1), jnp.float32)],
)


def _make_mid(K):
  def body(agg_ref, nd_ref, b_ref, ns_ref, w_ref, out_ref):
    a = agg_ref[0] + agg_ref[1]
    y = jnp.maximum(a * nd_ref[...] + b_ref[...], 0.0)
    out_ref[...] = jnp.dot(y * ns_ref[...], w_ref[...],
                           preferred_element_type=jnp.float32)

  return pl.pallas_call(
      body, out_shape=jax.ShapeDtypeStruct((NPAD, K), jnp.float32))


_mid_tc_h = _make_mid(H)


def _final_body(agg_ref, nd_ref, b_ref, out_ref):
  a = agg_ref[0, :, 0:OUT] + agg_ref[1, :, 0:OUT]
  out_ref[...] = a * nd_ref[...] + b_ref[...]


_final_tc = pl.pallas_call(
    _final_body,
    out_shape=jax.ShapeDtypeStruct((NPAD, OUT), jnp.float32),
)


def kernel(edge_index, W1, b1, W2, b2):
  src = edge_index[0].astype(jnp.int32)
  dst = edge_index[1].astype(jnp.int32)
  zeros_nh = jnp.zeros((NPAD, H), jnp.float32)
  zeros_c = jnp.zeros((CW,), jnp.float32)
  W2p = jnp.pad(W2, ((0, 0), (0, H - OUT)))

  sd = jnp.concatenate([src, dst])
  degs, csum = _prep_kernel(sd, zeros_c)
  dsrc2 = degs[:NPAD].reshape(NPAD, 1)
  ddst2 = degs[NPAD:].reshape(NPAD, 1)
  c0 = csum[:NPAD * H].reshape(NPAD, H)
  c1 = csum[NPAD * H:].reshape(NPAD, H)
  b1r = b1.reshape(1, H)
  b2r = b2.reshape(1, OUT)

  hw, ns2, nd2 = _l1mid_tc(c0, c1, dsrc2, ddst2, b1r, W1)
  agg = _agg_h(hw, src, dst, zeros_nh)
  hw = _mid_tc_h(agg, nd2, b1r, ns2, W1)
  agg = _agg_h(hw, src, dst, zeros_nh)
  hw2 = _mid_tc_h(agg, nd2, b1r, ns2, W2p)
  agg2 = _agg_h(hw2, src, dst, zeros_nh)
  out = _final_tc(agg2, nd2, b2r)
  return out[:N]


# speedup vs baseline: 10.6993x; 1.0012x over previous
import functools

import jax
import jax.numpy as jnp
from jax import lax
from jax.experimental import pallas as pl
from jax.experimental.pallas import tpu as pltpu
from jax.experimental.pallas import tpu_sc as plsc

N = 10000
E = 320000
H = 128
OUT = 64
NPAD = 10240

NC = 2
NS = 16
NW = NC * NS
RPT = NPAD // NS

EPW = E // NW
ACH = 80
ANCH = EPW // ACH
ASB = 125
ANSB = ANCH // ASB

EPT = E // NS
DBLK = 2000
CW = NPAD * H + 2048
CPT = NPAD * H // NS

_MESH = plsc.VectorSubcoreMesh(core_axis_name="c", subcore_axis_name="s")


def _prep_body(sd_hbm, zc_hbm, degs_hbm, csum_hbm,
               dstage_v, hist_v, part_v, cstage_s, cstage_d, ci0, ones_v,
               hstage, cacc, sem):
  cid = lax.axis_index("c")
  sid = lax.axis_index("s")
  ones_v[...] = jnp.ones((16,), jnp.float32)

  r0 = pl.multiple_of(sid * RPT, 8)
  z0 = pl.multiple_of(sid * (CW // NS), 8)
  pltpu.sync_copy(zc_hbm.at[pl.ds(z0, CW // NS)], cacc.at[pl.ds(z0, CW // NS)])

  doff = pl.multiple_of(cid * E + sid * EPT, 8)
  wid = cid * NS + sid
  coff = pl.multiple_of(wid * EPW, 8)

  def hzero(i, _):
    hist_v[pl.ds(i * 16, 16)] = jnp.zeros((16,), jnp.int32)
    return _

  lax.fori_loop(0, NPAD // 16, hzero, None)

  def dblk(blk, _):
    off = pl.multiple_of(doff + blk * DBLK, 8)
    pltpu.sync_copy(sd_hbm.at[pl.ds(off, DBLK)], dstage_v)

    def dgrp(g, _):
      for k in range(5):
        idx = dstage_v[pl.ds((g * 5 + k) * 16, 16)]
        cnt, last = plsc.scan_count(idx)
        plsc.addupdate_scatter(hist_v, [idx], cnt, mask=last)
      return _

    lax.fori_loop(0, DBLK // 80, dgrp, None)
    return _

  lax.fori_loop(0, EPT // DBLK, dblk, None)

  pltpu.sync_copy(hist_v, hstage.at[sid])
  plsc.subcore_barrier()
  pltpu.sync_copy(hstage.at[0].at[pl.ds(r0, RPT)], hist_v.at[pl.ds(0, RPT)])
  for b in range(1, NS):
    pltpu.sync_copy(hstage.at[b].at[pl.ds(r0, RPT)], part_v)
    for g in range(RPT // 16):
      sl = pl.ds(g * 16, 16)
      hist_v[sl] = hist_v[sl] + part_v[sl]
  pltpu.sync_copy(hist_v.at[pl.ds(0, RPT)],
                  degs_hbm.at[pl.ds(pl.multiple_of(cid * NPAD + r0, 8), RPT)])

  def cblk(blk, _):
    soff = pl.multiple_of(coff + blk * DBLK, 8)
    pltpu.sync_copy(sd_hbm.at[pl.ds(soff, DBLK)], cstage_s)
    pltpu.sync_copy(sd_hbm.at[pl.ds(soff + E, DBLK)], cstage_d)

    def cgrp(g, _):
      sl = pl.ds(g * 16, 16)
      s = cstage_s[sl]
      minv = lax.reduce_min(s, (0,))

      @pl.when(minv < H)
      def _():
        d = cstage_d[sl]
        ci0[...] = jnp.where(s < H, (d << 7) + s, (NPAD * H) + (s & 127))
        pltpu.sync_copy(ones_v, cacc.at[ci0], add=True)
      return _

    lax.fori_loop(0, DBLK // 16, cgrp, None)
    return _

  lax.fori_loop(0, EPW // DBLK, cblk, None)
  plsc.subcore_barrier()

  cd0 = pl.multiple_of(sid * CPT, 8)
  pltpu.sync_copy(
      cacc.at[pl.ds(cd0, CPT)],
      csum_hbm.at[pl.ds(pl.multiple_of(cid * (NPAD * H) + cd0, 8), CPT)])


_prep_kernel = functools.partial(
    pl.kernel,
    out_type=[jax.ShapeDtypeStruct((2 * NPAD,), jnp.int32),
              jax.ShapeDtypeStruct((2 * NPAD * H,), jnp.float32)],
    mesh=_MESH,
    compiler_params=pltpu.CompilerParams(needs_layout_passes=False),
    scratch_types=[
        pltpu.VMEM((DBLK,), jnp.int32),
        pltpu.VMEM((NPAD,), jnp.int32),
        pltpu.VMEM((RPT,), jnp.int32),
        pltpu.VMEM((DBLK,), jnp.int32),
        pltpu.VMEM((DBLK,), jnp.int32),
        pltpu.VMEM((16,), jnp.int32),
        pltpu.VMEM((16,), jnp.float32),
        pltpu.VMEM_SHARED((NS, NPAD), jnp.int32),
        pltpu.VMEM_SHARED((CW,), jnp.float32),
        pltpu.SemaphoreType.DMA,
    ],
)(_prep_body)


def _make_agg(F):
  def body(hw_hbm, src_hbm, dst_hbm, zeros_hbm, out_hbm,
           sstage, dstage, didx0, didx1, rows0, rows1, acc,
           sg0, sg1, ss0, ss1):
    cid = lax.axis_index("c")
    sid = lax.axis_index("s")
    wid = cid * NS + sid
    r0 = pl.multiple_of(sid * RPT, 8)
    pltpu.sync_copy(zeros_hbm.at[pl.ds(r0, RPT)], acc.at[pl.ds(r0, RPT)])
    plsc.subcore_barrier()
    base = wid * EPW

    def vcopy(dref, j):
      for o in sorted(set(list(range(0, ACH - 15, 16)) + [ACH - 16])):
        dref[pl.ds(o, 16)] = dstage[pl.ds(j * ACH + o, 16)]

    def g_desc(j, rbuf, sem):
      return pltpu.make_async_copy(
          hw_hbm.at[sstage.at[pl.ds(j * ACH, ACH)]], rbuf, sem)

    def s_desc(rbuf, dref, sem):
      return pltpu.make_async_copy(rbuf, acc.at[dref], sem)

    def sblock(sb, _):
      off = pl.multiple_of(base + sb * (ASB * ACH), 8)
      pltpu.sync_copy(src_hbm.at[pl.ds(off, ASB * ACH)], sstage)
      pltpu.sync_copy(dst_hbm.at[pl.ds(off, ASB * ACH)], dstage)
      vcopy(didx0, 0)
      g_desc(0, rows0, sg0).start()

      def pair(t, _):
        j0 = 2 * t
        j1 = j0 + 1
        g_desc(j1, rows1, sg1).start()
        g_desc(j0, rows0, sg0).wait()

        @pl.when(t > 0)
        def _():
          s_desc(rows1, didx1, ss1).wait()

        vcopy(didx1, j1)
        pltpu.async_copy(rows0, acc.at[didx0], ss0, add=True)
        g_desc(j1, rows1, sg1).wait()
        s_desc(rows0, didx0, ss0).wait()
        vcopy(didx0, j0 + 2)
        g_desc(j0 + 2, rows0, sg0).start()
        pltpu.async_copy(rows1, acc.at[didx1], ss1, add=True)
        return _

      lax.fori_loop(0, ASB // 2, pair, None)
      g_desc(ASB - 1, rows0, sg0).wait()
      s_desc(rows1, didx1, ss1).wait()
      pltpu.async_copy(rows0, acc.at[didx0], ss0, add=True)
      s_desc(rows0, didx0, ss0).wait()
      return _

    lax.fori_loop(0, ANSB, sblock, None)
    plsc.subcore_barrier()
    pltpu.sync_copy(acc.at[pl.ds(r0, RPT)],
                    out_hbm.at[cid].at[pl.ds(r0, RPT)])

  return pl.kernel(
      body,
      out_type=jax.ShapeDtypeStruct((NC, NPAD, F), jnp.float32),
      mesh=_MESH,
      scratch_types=[
          pltpu.VMEM((ASB * ACH,), jnp.int32),
          pltpu.VMEM((ASB * ACH,), jnp.int32),
          pltpu.VMEM((ACH,), jnp.int32),
          pltpu.VMEM((ACH,), jnp.int32),
          pltpu.VMEM((ACH, F), jnp.float32),
          pltpu.VMEM((ACH, F), jnp.float32),
          pltpu.VMEM_SHARED((NPAD, F), jnp.float32),
          pltpu.SemaphoreType.DMA,
          pltpu.SemaphoreType.DMA,
          pltpu.SemaphoreType.DMA,
          pltpu.SemaphoreType.DMA,
      ],
  )


_agg_h = _make_agg(H)


def _l1mid_body(c0_ref, c1_ref, dsrc_ref, ddst_ref, b_ref, w_ref,
                out_ref, ns_ref, nd_ref):
  ns = lax.rsqrt(jnp.maximum(dsrc_ref[...].astype(jnp.float32), 1.0))
  nd = lax.rsqrt(jnp.maximum(ddst_ref[...].astype(jnp.float32), 1.0))
  ns_ref[...] = ns
  nd_ref[...] = nd
  c = c0_ref[...] + c1_ref[...]
  w1s = w_ref[...] * ns[0:H]
  agg = jnp.dot(c, w1s, preferred_element_type=jnp.float32)
  y = jnp.maximum(agg * nd + b_ref[...], 0.0)
  out_ref[...] = jnp.dot(y * ns, w_ref[...],
                         preferred_element_type=jnp.float32)


_l1mid_tc = pl.pallas_call(
    _l1mid_body,
    out_shape=[jax.ShapeDtypeStruct((NPAD, H), jnp.float32),
               jax.ShapeDtypeStruct((NPAD, 1), jnp.float32),
               jax.ShapeDtypeStruct((NPAD, 1), jnp.float32)],
)


def _make_mid(K):
  def body(agg_ref, nd_ref, b_ref, ns_ref, w_ref, out_ref):
    a = agg_ref[0] + agg_ref[1]
    y = jnp.maximum(a * nd_ref[...] + b_ref[...], 0.0)
    out_ref[...] = jnp.dot(y * ns_ref[...], w_ref[...],
                           preferred_element_type=jnp.float32)

  return pl.pallas_call(
      body, out_shape=jax.ShapeDtypeStruct((NPAD, K), jnp.float32))


_mid_tc_h = _make_mid(H)


def _final_body(agg_ref, nd_ref, b_ref, out_ref):
  a = agg_ref[0, :, 0:OUT] + agg_ref[1, :, 0:OUT]
  out_ref[...] = a * nd_ref[...] + b_ref[...]


_final_tc = pl.pallas_call(
    _final_body,
    out_shape=jax.ShapeDtypeStruct((NPAD, OUT), jnp.float32),
)


def kernel(edge_index, W1, b1, W2, b2):
  src = edge_index[0].astype(jnp.int32)
  dst = edge_index[1].astype(jnp.int32)
  zeros_nh = jnp.zeros((NPAD, H), jnp.float32)
  zeros_c = jnp.zeros((CW,), jnp.float32)
  W2p = jnp.pad(W2, ((0, 0), (0, H - OUT)))

  sd = jnp.concatenate([src, dst])
  degs, csum = _prep_kernel(sd, zeros_c)
  dsrc2 = degs[:NPAD].reshape(NPAD, 1)
  ddst2 = degs[NPAD:].reshape(NPAD, 1)
  c0 = csum[:NPAD * H].reshape(NPAD, H)
  c1 = csum[NPAD * H:].reshape(NPAD, H)
  b1r = b1.reshape(1, H)
  b2r = b2.reshape(1, OUT)

  hw, ns2, nd2 = _l1mid_tc(c0, c1, dsrc2, ddst2, b1r, W1)
  agg = _agg_h(hw, src, dst, zeros_nh)
  hw = _mid_tc_h(agg, nd2, b1r, ns2, W1)
  agg = _agg_h(hw, src, dst, zeros_nh)
  hw2 = _mid_tc_h(agg, nd2, b1r, ns2, W2p)
  agg2 = _agg_h(hw2, src, dst, zeros_nh)
  out = _final_tc(agg2, nd2, b2r)
  return out[:N]
